# Initial kernel scaffold; baseline (speedup 1.0000x reference)
import jax
import jax.numpy as jnp
from jax.experimental import pallas as pl


def kernel(x, edge_index, edge_attr, params):
    raise NotImplementedError("write your pallas kernel here")


# speedup vs baseline: 4.4303x; 4.4303x over previous
import functools

import jax
import jax.numpy as jnp
from jax import lax
from jax.experimental import pallas as pl

N = 10000
E = 160000
FEAT = 256
C = 64
H = 8
HC = H * C


def _linbn_body(h_ref, w_ref, b_ref, g_ref, be_ref, o_ref):
    hh = lax.dot_general(h_ref[...], w_ref[...],
                         (((1,), (1,)), ((), ())),
                         preferred_element_type=jnp.float32)
    hh = hh + b_ref[...]
    m = jnp.mean(hh, axis=0, keepdims=True)
    v = jnp.mean((hh - m) ** 2, axis=0, keepdims=True)
    o_ref[...] = jnp.maximum((hh - m) / jnp.sqrt(v + 1e-5) * g_ref[...]
                             + be_ref[...], 0.0)


def _lin_bn_relu(h, W, b, g, be):
    n, _ = h.shape
    co = W.shape[0]
    return pl.pallas_call(
        _linbn_body,
        out_shape=jax.ShapeDtypeStruct((n, co), jnp.float32),
    )(h, W, b.reshape(1, co), g.reshape(1, co), be.reshape(1, co))


def _proj_body(h_ref, wl_ref, bl_ref, wr_ref, br_ref, xl_ref, xr_ref):
    h = h_ref[...]
    xl_ref[...] = lax.dot_general(h, wl_ref[...], (((1,), (1,)), ((), ())),
                                  preferred_element_type=jnp.float32) + bl_ref[...]
    xr_ref[...] = lax.dot_general(h, wr_ref[...], (((1,), (1,)), ((), ())),
                                  preferred_element_type=jnp.float32) + br_ref[...]


_RB = 2000


def _gat_proj(h, p):
    n = h.shape[0]
    nb = n // _RB
    return pl.pallas_call(
        _proj_body,
        grid=(nb,),
        in_specs=[
            pl.BlockSpec((_RB, C), lambda i: (i, 0)),
            pl.BlockSpec((HC, C), lambda i: (0, 0)),
            pl.BlockSpec((1, HC), lambda i: (0, 0)),
            pl.BlockSpec((HC, C), lambda i: (0, 0)),
            pl.BlockSpec((1, HC), lambda i: (0, 0)),
        ],
        out_specs=(pl.BlockSpec((_RB, HC), lambda i: (i, 0)),
                   pl.BlockSpec((_RB, HC), lambda i: (i, 0))),
        out_shape=(jax.ShapeDtypeStruct((n, HC), jnp.float32),
                   jax.ShapeDtypeStruct((n, HC), jnp.float32)),
    )(h, p["Wl"], p["bl"].reshape(1, HC), p["Wr"], p["br"].reshape(1, HC))


def _self_score_body(xl_ref, xr_ref, mea_ref, we_ref, attf_ref, sel_ref,
                     a_ref):
    ee = lax.dot_general(mea_ref[...], we_ref[...], (((1,), (1,)), ((), ())),
                         preferred_element_type=jnp.float32)
    z = xl_ref[...] + xr_ref[...] + ee
    z = jnp.where(z >= 0.0, z, 0.2 * z)
    za = z * attf_ref[...]
    a_ref[...] = lax.dot_general(za, sel_ref[...], (((1,), (0,)), ((), ())),
                                 preferred_element_type=jnp.float32)


def _self_scores(xl, xr, mean_ea, We, att):
    attf = att.reshape(1, HC)
    sel = jnp.zeros((HC, H), jnp.float32)
    sel = sel.at[jnp.arange(HC), jnp.arange(HC) // C].set(1.0)
    nb = N // _RB
    return pl.pallas_call(
        _self_score_body,
        grid=(nb,),
        in_specs=[
            pl.BlockSpec((_RB, HC), lambda i: (i, 0)),
            pl.BlockSpec((_RB, HC), lambda i: (i, 0)),
            pl.BlockSpec((_RB, 2), lambda i: (i, 0)),
            pl.BlockSpec((HC, 2), lambda i: (0, 0)),
            pl.BlockSpec((1, HC), lambda i: (0, 0)),
            pl.BlockSpec((HC, H), lambda i: (0, 0)),
        ],
        out_specs=pl.BlockSpec((_RB, H), lambda i: (i, 0)),
        out_shape=jax.ShapeDtypeStruct((N, H), jnp.float32),
    )(xl, xr, mean_ea, We, attf, sel)


def _final_body(h_ref, w_ref, b_ref, g_ref, be_ref, wf_ref, bf_ref, o_ref):
    hh = lax.dot_general(h_ref[...], w_ref[...], (((1,), (1,)), ((), ())),
                         preferred_element_type=jnp.float32) + b_ref[...]
    m = jnp.mean(hh, axis=0, keepdims=True)
    v = jnp.mean((hh - m) ** 2, axis=0, keepdims=True)
    hh = jnp.maximum((hh - m) / jnp.sqrt(v + 1e-5) * g_ref[...] + be_ref[...],
                     0.0)
    lg = lax.dot_general(hh, wf_ref[...], (((1,), (1,)), ((), ())),
                         preferred_element_type=jnp.float32) + bf_ref[...]
    mx = jnp.max(lg, axis=1, keepdims=True)
    el = jnp.exp(lg - mx)
    sm = el / jnp.sum(el, axis=1, keepdims=True)
    o_ref[...] = sm[:, 1:]


def _final_stage(h, W, b, g, be, WF, bF):
    return pl.pallas_call(
        _final_body,
        out_shape=jax.ShapeDtypeStruct((N, 1), jnp.float32),
    )(h, W, b.reshape(1, C), g.reshape(1, C), be.reshape(1, C),
      WF, bF.reshape(1, 2))


def _gat_layer(h, src, dst, ea, p, mask):
    maskf = mask.astype(jnp.float32)
    mea = ea * maskf[:, None]
    sum_ea = jax.ops.segment_sum(mea, dst, num_segments=N)
    cnt = jax.ops.segment_sum(maskf, dst, num_segments=N)
    mean_ea = sum_ea / jnp.maximum(cnt, 1.0)[:, None]

    xl, xr = _gat_proj(h, p)

    a_self = _self_scores(xl, xr, mean_ea, p["We"], p["att"])

    ee = ea @ p["We"].T
    z = xl[src] + xr[dst] + ee
    z = jnp.where(z >= 0.0, z, 0.2 * z)
    a_e = (z.reshape(E, H, C) * p["att"]).sum(-1)
    a_e = jnp.where(mask[:, None], a_e, -jnp.inf)

    amax = jnp.maximum(jax.ops.segment_max(a_e, dst, num_segments=N,
                                           indices_are_sorted=False), a_self)
    p_e = jnp.exp(a_e - amax[dst])
    p_self = jnp.exp(a_self - amax)
    asum = jax.ops.segment_sum(p_e, dst, num_segments=N) + p_self

    alpha_e = p_e / (asum[dst] + 1e-16)
    alpha_self = p_self / (asum + 1e-16)

    contrib = (xl[src].reshape(E, H, C) * alpha_e[..., None]).reshape(E, HC)
    contrib = contrib * maskf[:, None]
    agg = jax.ops.segment_sum(contrib, dst, num_segments=N)
    self_c = (xl.reshape(N, H, C) * alpha_self[..., None]).reshape(N, HC)
    out = agg + self_c + p["bias"]
    return jnp.maximum(out, 0.0)


def kernel(x, edge_index, edge_attr, params):
    src = edge_index[0]
    dst = edge_index[1]
    a0 = edge_attr[:, 0]
    fea = edge_attr[:, 1:]
    m1 = a0 >= 0
    m2 = a0 <= 0
    mall = jnp.ones((E,), dtype=bool)
    p = params

    h = _lin_bn_relu(x, p["W0"], p["b0"], p["g0"], p["be0"])
    h = _gat_layer(h, src, dst, fea, p["gat1"], m1)
    h = _lin_bn_relu(h, p["W1"], p["b1"], p["g1"], p["be1"])
    h = _gat_layer(h, src, dst, fea, p["gat2"], m2)
    h = _lin_bn_relu(h, p["W2"], p["b2"], p["g2"], p["be2"])
    h = _gat_layer(h, src, dst, fea, p["gat3"], mall)
    return _final_stage(h, p["W3"], p["b3"], p["g3"], p["be3"],
                        p["WF"], p["bF"])


# speedup vs baseline: 4.9591x; 1.1194x over previous
import functools

import jax
import jax.numpy as jnp
from jax import lax
from jax.experimental import pallas as pl
from jax.experimental.pallas import tpu as pltpu
from jax.experimental.pallas import tpu_sc as plsc

N = 10000
E = 160000
FEAT = 256
C = 64
H = 8
HC = H * C


def _linbn_body(h_ref, w_ref, b_ref, g_ref, be_ref, o_ref):
    hh = lax.dot_general(h_ref[...], w_ref[...],
                         (((1,), (1,)), ((), ())),
                         preferred_element_type=jnp.float32)
    hh = hh + b_ref[...]
    m = jnp.mean(hh, axis=0, keepdims=True)
    v = jnp.mean((hh - m) ** 2, axis=0, keepdims=True)
    o_ref[...] = jnp.maximum((hh - m) / jnp.sqrt(v + 1e-5) * g_ref[...]
                             + be_ref[...], 0.0)


def _lin_bn_relu(h, W, b, g, be):
    n, _ = h.shape
    co = W.shape[0]
    return pl.pallas_call(
        _linbn_body,
        out_shape=jax.ShapeDtypeStruct((n, co), jnp.float32),
    )(h, W, b.reshape(1, co), g.reshape(1, co), be.reshape(1, co))


def _proj_body(h_ref, wl_ref, bl_ref, wr_ref, br_ref, xl_ref, xr_ref):
    h = h_ref[...]
    xl_ref[...] = lax.dot_general(h, wl_ref[...], (((1,), (1,)), ((), ())),
                                  preferred_element_type=jnp.float32) + bl_ref[...]
    xr_ref[...] = lax.dot_general(h, wr_ref[...], (((1,), (1,)), ((), ())),
                                  preferred_element_type=jnp.float32) + br_ref[...]


_RB = 2000


def _gat_proj(h, p):
    n = h.shape[0]
    nb = n // _RB
    return pl.pallas_call(
        _proj_body,
        grid=(nb,),
        in_specs=[
            pl.BlockSpec((_RB, C), lambda i: (i, 0)),
            pl.BlockSpec((HC, C), lambda i: (0, 0)),
            pl.BlockSpec((1, HC), lambda i: (0, 0)),
            pl.BlockSpec((HC, C), lambda i: (0, 0)),
            pl.BlockSpec((1, HC), lambda i: (0, 0)),
        ],
        out_specs=(pl.BlockSpec((_RB, HC), lambda i: (i, 0)),
                   pl.BlockSpec((_RB, HC), lambda i: (i, 0))),
        out_shape=(jax.ShapeDtypeStruct((n, HC), jnp.float32),
                   jax.ShapeDtypeStruct((n, HC), jnp.float32)),
    )(h, p["Wl"], p["bl"].reshape(1, HC), p["Wr"], p["br"].reshape(1, HC))


def _self_score_body(xl_ref, xr_ref, mea_ref, we_ref, attf_ref, sel_ref,
                     a_ref):
    ee = lax.dot_general(mea_ref[...], we_ref[...], (((1,), (1,)), ((), ())),
                         preferred_element_type=jnp.float32)
    z = xl_ref[...] + xr_ref[...] + ee
    z = jnp.where(z >= 0.0, z, 0.2 * z)
    za = z * attf_ref[...]
    a_ref[...] = lax.dot_general(za, sel_ref[...], (((1,), (0,)), ((), ())),
                                 preferred_element_type=jnp.float32)


def _self_scores(xl, xr, mean_ea, We, att):
    attf = att.reshape(1, HC)
    sel = jnp.zeros((HC, H), jnp.float32)
    sel = sel.at[jnp.arange(HC), jnp.arange(HC) // C].set(1.0)
    nb = N // _RB
    return pl.pallas_call(
        _self_score_body,
        grid=(nb,),
        in_specs=[
            pl.BlockSpec((_RB, HC), lambda i: (i, 0)),
            pl.BlockSpec((_RB, HC), lambda i: (i, 0)),
            pl.BlockSpec((_RB, 2), lambda i: (i, 0)),
            pl.BlockSpec((HC, 2), lambda i: (0, 0)),
            pl.BlockSpec((1, HC), lambda i: (0, 0)),
            pl.BlockSpec((HC, H), lambda i: (0, 0)),
        ],
        out_specs=pl.BlockSpec((_RB, H), lambda i: (i, 0)),
        out_shape=jax.ShapeDtypeStruct((N, H), jnp.float32),
    )(xl, xr, mean_ea, We, attf, sel)


def _final_body(h_ref, w_ref, b_ref, g_ref, be_ref, wf_ref, bf_ref, o_ref):
    hh = lax.dot_general(h_ref[...], w_ref[...], (((1,), (1,)), ((), ())),
                         preferred_element_type=jnp.float32) + b_ref[...]
    m = jnp.mean(hh, axis=0, keepdims=True)
    v = jnp.mean((hh - m) ** 2, axis=0, keepdims=True)
    hh = jnp.maximum((hh - m) / jnp.sqrt(v + 1e-5) * g_ref[...] + be_ref[...],
                     0.0)
    lg = lax.dot_general(hh, wf_ref[...], (((1,), (1,)), ((), ())),
                         preferred_element_type=jnp.float32) + bf_ref[...]
    mx = jnp.max(lg, axis=1, keepdims=True)
    el = jnp.exp(lg - mx)
    sm = el / jnp.sum(el, axis=1, keepdims=True)
    o_ref[...] = sm[:, 1:]


def _final_stage(h, W, b, g, be, WF, bF):
    return pl.pallas_call(
        _final_body,
        out_shape=jax.ShapeDtypeStruct((N, 1), jnp.float32),
    )(h, W, b.reshape(1, C), g.reshape(1, C), be.reshape(1, C),
      WF, bF.reshape(1, 2))


_NW = 32
_EPW = E // _NW
_CH = 40
_NCHUNK = _EPW // _CH


def _zsum_body(xl_hbm, xr_hbm, src_hbm, dst_hbm, z_hbm,
               sidx, didx, xlb, xrb, sem1, sem2):
    w = lax.axis_index("s") * 2 + lax.axis_index("c")
    base = w * _EPW
    pltpu.sync_copy(src_hbm.at[pl.ds(base, _EPW)], sidx)
    pltpu.sync_copy(dst_hbm.at[pl.ds(base, _EPW)], didx)

    def chunk(ci, carry):
        o = ci * _CH
        cp1 = pltpu.async_copy(xl_hbm.at[sidx.at[pl.ds(o, _CH)]], xlb, sem1)
        cp2 = pltpu.async_copy(xr_hbm.at[didx.at[pl.ds(o, _CH)]], xrb, sem2)
        cp1.wait()
        cp2.wait()

        def row(r, c2):
            for j in range(HC // 16):
                xlb[r, pl.ds(j * 16, 16)] = (xlb[r, pl.ds(j * 16, 16)]
                                             + xrb[r, pl.ds(j * 16, 16)])
            return c2

        lax.fori_loop(0, _CH, row, 0, unroll=False)
        pltpu.sync_copy(xlb, z_hbm.at[pl.ds(base + o, _CH)])
        return carry

    lax.fori_loop(0, _NCHUNK, chunk, 0, unroll=False)


def _zsum_sc(xl, xr, ssrc, sdst):
    mesh = plsc.VectorSubcoreMesh(core_axis_name="c", subcore_axis_name="s")
    k = functools.partial(
        pl.kernel,
        mesh=mesh,
        out_type=jax.ShapeDtypeStruct((E, HC), jnp.float32),
        scratch_types=[
            pltpu.VMEM((_EPW,), jnp.int32),
            pltpu.VMEM((_EPW,), jnp.int32),
            pltpu.VMEM((_CH, HC), jnp.float32),
            pltpu.VMEM((_CH, HC), jnp.float32),
            pltpu.SemaphoreType.DMA,
            pltpu.SemaphoreType.DMA,
        ],
    )(_zsum_body)
    return k(xl, xr, ssrc, sdst)


_EB = 2000


def _escore_body(z_ref, ea_ref, we_ref, attf_ref, sel_ref, m_ref, o_ref):
    ee = lax.dot_general(ea_ref[...], we_ref[...], (((1,), (1,)), ((), ())),
                         preferred_element_type=jnp.float32)
    z = z_ref[...] + ee
    z = jnp.where(z >= 0.0, z, 0.2 * z)
    a = lax.dot_general(z * attf_ref[...], sel_ref[...],
                        (((1,), (0,)), ((), ())),
                        preferred_element_type=jnp.float32)
    o_ref[...] = jnp.where(m_ref[...] > 0.0, a, -jnp.inf)


def _edge_scores(zsum, sea, We, att, maskf):
    attf = att.reshape(1, HC)
    sel = jnp.zeros((HC, H), jnp.float32)
    sel = sel.at[jnp.arange(HC), jnp.arange(HC) // C].set(1.0)
    nb = E // _EB
    return pl.pallas_call(
        _escore_body,
        grid=(nb,),
        in_specs=[
            pl.BlockSpec((_EB, HC), lambda i: (i, 0)),
            pl.BlockSpec((_EB, 2), lambda i: (i, 0)),
            pl.BlockSpec((HC, 2), lambda i: (0, 0)),
            pl.BlockSpec((1, HC), lambda i: (0, 0)),
            pl.BlockSpec((HC, H), lambda i: (0, 0)),
            pl.BlockSpec((_EB, 1), lambda i: (i, 0)),
        ],
        out_specs=pl.BlockSpec((_EB, H), lambda i: (i, 0)),
        out_shape=jax.ShapeDtypeStruct((E, H), jnp.float32),
    )(zsum, sea, We, attf, sel, maskf.reshape(E, 1))


def _gat_layer(h, ssrc, sdst, sea, p, smaskf):
    mea = sea * smaskf[:, None]
    sum_ea = jax.ops.segment_sum(mea, sdst, num_segments=N,
                                 indices_are_sorted=True)
    cnt = jax.ops.segment_sum(smaskf, sdst, num_segments=N,
                              indices_are_sorted=True)
    mean_ea = sum_ea / jnp.maximum(cnt, 1.0)[:, None]

    xl, xr = _gat_proj(h, p)

    a_self = _self_scores(xl, xr, mean_ea, p["We"], p["att"])

    zsum = _zsum_sc(xl, xr, ssrc, sdst)
    a_e = _edge_scores(zsum, sea, p["We"], p["att"], smaskf)

    amax = jnp.maximum(jax.ops.segment_max(a_e, sdst, num_segments=N,
                                           indices_are_sorted=True), a_self)
    p_e = jnp.exp(a_e - amax[sdst])
    p_self = jnp.exp(a_self - amax)
    asum = jax.ops.segment_sum(p_e, sdst, num_segments=N,
                               indices_are_sorted=True) + p_self

    alpha_e = p_e / (asum[sdst] + 1e-16)
    alpha_self = p_self / (asum + 1e-16)

    contrib = (xl[ssrc].reshape(E, H, C) * alpha_e[..., None]).reshape(E, HC)
    contrib = contrib * smaskf[:, None]
    agg = jax.ops.segment_sum(contrib, sdst, num_segments=N,
                              indices_are_sorted=True)
    self_c = (xl.reshape(N, H, C) * alpha_self[..., None]).reshape(N, HC)
    out = agg + self_c + p["bias"]
    return jnp.maximum(out, 0.0)


def kernel(x, edge_index, edge_attr, params):
    src = edge_index[0]
    dst = edge_index[1]
    a0 = edge_attr[:, 0]
    fea = edge_attr[:, 1:]
    p = params

    perm = jnp.argsort(dst)
    ssrc = src[perm]
    sdst = dst[perm]
    sea = fea[perm]
    sa0 = a0[perm]
    m1 = (sa0 >= 0).astype(jnp.float32)
    m2 = (sa0 <= 0).astype(jnp.float32)
    mall = jnp.ones((E,), jnp.float32)

    h = _lin_bn_relu(x, p["W0"], p["b0"], p["g0"], p["be0"])
    h = _gat_layer(h, ssrc, sdst, sea, p["gat1"], m1)
    h = _lin_bn_relu(h, p["W1"], p["b1"], p["g1"], p["be1"])
    h = _gat_layer(h, ssrc, sdst, sea, p["gat2"], m2)
    h = _lin_bn_relu(h, p["W2"], p["b2"], p["g2"], p["be2"])
    h = _gat_layer(h, ssrc, sdst, sea, p["gat3"], mall)
    return _final_stage(h, p["W3"], p["b3"], p["g3"], p["be3"],
                        p["WF"], p["bF"])


# speedup vs baseline: 7.3574x; 1.4836x over previous
import functools

import jax
import jax.numpy as jnp
from jax import lax
from jax.experimental import pallas as pl
from jax.experimental.pallas import tpu as pltpu
from jax.experimental.pallas import tpu_sc as plsc

N = 10000
E = 160000
FEAT = 256
C = 64
H = 8
HC = H * C


def _linbn_body(h_ref, w_ref, b_ref, g_ref, be_ref, o_ref):
    hh = lax.dot_general(h_ref[...], w_ref[...],
                         (((1,), (1,)), ((), ())),
                         preferred_element_type=jnp.float32)
    hh = hh + b_ref[...]
    m = jnp.mean(hh, axis=0, keepdims=True)
    v = jnp.mean((hh - m) ** 2, axis=0, keepdims=True)
    o_ref[...] = jnp.maximum((hh - m) / jnp.sqrt(v + 1e-5) * g_ref[...]
                             + be_ref[...], 0.0)


def _lin_bn_relu(h, W, b, g, be):
    n, _ = h.shape
    co = W.shape[0]
    return pl.pallas_call(
        _linbn_body,
        out_shape=jax.ShapeDtypeStruct((n, co), jnp.float32),
    )(h, W, b.reshape(1, co), g.reshape(1, co), be.reshape(1, co))


def _proj_body(h_ref, wl_ref, bl_ref, wr_ref, br_ref, xl_ref, xr_ref):
    h = h_ref[...]
    xl_ref[...] = lax.dot_general(h, wl_ref[...], (((1,), (1,)), ((), ())),
                                  preferred_element_type=jnp.float32) + bl_ref[...]
    xr_ref[...] = lax.dot_general(h, wr_ref[...], (((1,), (1,)), ((), ())),
                                  preferred_element_type=jnp.float32) + br_ref[...]


_RB = 2000


def _gat_proj(h, p):
    n = h.shape[0]
    nb = n // _RB
    return pl.pallas_call(
        _proj_body,
        grid=(nb,),
        in_specs=[
            pl.BlockSpec((_RB, C), lambda i: (i, 0)),
            pl.BlockSpec((HC, C), lambda i: (0, 0)),
            pl.BlockSpec((1, HC), lambda i: (0, 0)),
            pl.BlockSpec((HC, C), lambda i: (0, 0)),
            pl.BlockSpec((1, HC), lambda i: (0, 0)),
        ],
        out_specs=(pl.BlockSpec((_RB, HC), lambda i: (i, 0)),
                   pl.BlockSpec((_RB, HC), lambda i: (i, 0))),
        out_shape=(jax.ShapeDtypeStruct((n, HC), jnp.float32),
                   jax.ShapeDtypeStruct((n, HC), jnp.float32)),
    )(h, p["Wl"], p["bl"].reshape(1, HC), p["Wr"], p["br"].reshape(1, HC))


def _self_score_body(xl_ref, xr_ref, mea_ref, we_ref, attf_ref, sel_ref,
                     a_ref):
    ee = lax.dot_general(mea_ref[...], we_ref[...], (((1,), (1,)), ((), ())),
                         preferred_element_type=jnp.float32)
    z = xl_ref[...] + xr_ref[...] + ee
    z = jnp.where(z >= 0.0, z, 0.2 * z)
    za = z * attf_ref[...]
    a_ref[...] = lax.dot_general(za, sel_ref[...], (((1,), (0,)), ((), ())),
                                 preferred_element_type=jnp.float32)


def _self_scores(xl, xr, mean_ea, We, att):
    attf = att.reshape(1, HC)
    sel = jnp.zeros((HC, H), jnp.float32)
    sel = sel.at[jnp.arange(HC), jnp.arange(HC) // C].set(1.0)
    nb = N // _RB
    return pl.pallas_call(
        _self_score_body,
        grid=(nb,),
        in_specs=[
            pl.BlockSpec((_RB, HC), lambda i: (i, 0)),
            pl.BlockSpec((_RB, HC), lambda i: (i, 0)),
            pl.BlockSpec((_RB, 2), lambda i: (i, 0)),
            pl.BlockSpec((HC, 2), lambda i: (0, 0)),
            pl.BlockSpec((1, HC), lambda i: (0, 0)),
            pl.BlockSpec((HC, H), lambda i: (0, 0)),
        ],
        out_specs=pl.BlockSpec((_RB, H), lambda i: (i, 0)),
        out_shape=jax.ShapeDtypeStruct((N, H), jnp.float32),
    )(xl, xr, mean_ea, We, attf, sel)


def _final_body(h_ref, w_ref, b_ref, g_ref, be_ref, wf_ref, bf_ref, o_ref):
    hh = lax.dot_general(h_ref[...], w_ref[...], (((1,), (1,)), ((), ())),
                         preferred_element_type=jnp.float32) + b_ref[...]
    m = jnp.mean(hh, axis=0, keepdims=True)
    v = jnp.mean((hh - m) ** 2, axis=0, keepdims=True)
    hh = jnp.maximum((hh - m) / jnp.sqrt(v + 1e-5) * g_ref[...] + be_ref[...],
                     0.0)
    lg = lax.dot_general(hh, wf_ref[...], (((1,), (1,)), ((), ())),
                         preferred_element_type=jnp.float32) + bf_ref[...]
    mx = jnp.max(lg, axis=1, keepdims=True)
    el = jnp.exp(lg - mx)
    sm = el / jnp.sum(el, axis=1, keepdims=True)
    o_ref[...] = sm[:, 1:]


def _final_stage(h, W, b, g, be, WF, bF):
    return pl.pallas_call(
        _final_body,
        out_shape=jax.ShapeDtypeStruct((N, 1), jnp.float32),
    )(h, W, b.reshape(1, C), g.reshape(1, C), be.reshape(1, C),
      WF, bF.reshape(1, 2))


_NW = 32
_EPW = E // _NW
_CH = 40
_NCHUNK = _EPW // _CH


def _zsum_body(xl_hbm, xr_hbm, src_hbm, dst_hbm, z_hbm,
               sidx, didx, xlb, xrb, sem1, sem2):
    w = lax.axis_index("s") * 2 + lax.axis_index("c")
    base = w * _EPW
    pltpu.sync_copy(src_hbm.at[pl.ds(base, _EPW)], sidx)
    pltpu.sync_copy(dst_hbm.at[pl.ds(base, _EPW)], didx)

    def chunk(ci, carry):
        o = ci * _CH
        cp1 = pltpu.async_copy(xl_hbm.at[sidx.at[pl.ds(o, _CH)]], xlb, sem1)
        cp2 = pltpu.async_copy(xr_hbm.at[didx.at[pl.ds(o, _CH)]], xrb, sem2)
        cp1.wait()
        cp2.wait()

        def row(r, c2):
            for j in range(HC // 16):
                xlb[r, pl.ds(j * 16, 16)] = (xlb[r, pl.ds(j * 16, 16)]
                                             + xrb[r, pl.ds(j * 16, 16)])
            return c2

        lax.fori_loop(0, _CH, row, 0, unroll=False)
        pltpu.sync_copy(xlb, z_hbm.at[pl.ds(base + o, _CH)])
        return carry

    lax.fori_loop(0, _NCHUNK, chunk, 0, unroll=False)


_EPAD = E + 64


def _zsum_sc(xl, xr, ssrc, sdst):
    mesh = plsc.VectorSubcoreMesh(core_axis_name="c", subcore_axis_name="s")
    k = functools.partial(
        pl.kernel,
        mesh=mesh,
        compiler_params=pltpu.CompilerParams(needs_layout_passes=False),
        out_type=jax.ShapeDtypeStruct((_EPAD, HC), jnp.float32),
        scratch_types=[
            pltpu.VMEM((_EPW,), jnp.int32),
            pltpu.VMEM((_EPW,), jnp.int32),
            pltpu.VMEM((_CH, HC), jnp.float32),
            pltpu.VMEM((_CH, HC), jnp.float32),
            pltpu.SemaphoreType.DMA,
            pltpu.SemaphoreType.DMA,
        ],
    )(_zsum_body)
    return k(xl, xr, ssrc, sdst)


_NR = 160
_RN = 64
_RPW = _NR // _NW
_CHS = 24
_NPAD = _NR * _RN
_I16 = lambda: lax.iota(jnp.int32, 16)


def _bcast16(v):
    return jnp.full((16,), v, jnp.int32)


def _scalar(ref, i):
    v = plsc.load_gather(ref, [_bcast16(i)])
    return lax.reduce_max(v, axes=(0,))


def _range_bounds(offsbuf, r):
    lo = _scalar(offsbuf, r)
    hi = _scalar(offsbuf, r + 1)
    return lo, hi


def _mea_body(mepk_hbm, offs_hbm, macc_hbm, offsbuf, mbuf, macc, sem):
    w = lax.axis_index("s") * 2 + lax.axis_index("c")
    pltpu.sync_copy(offs_hbm, offsbuf)
    m3 = _I16() < 3
    zeros = jnp.zeros((16,), jnp.float32)
    for half in range(_RPW):
        r = _RPW * w + half
        nodebase = r * _RN
        lo, hi = _range_bounds(offsbuf, r)

        def zrow(n, c):
            plsc.store_scatter(macc, [_bcast16(n), _I16()], zeros)
            return c
        lax.fori_loop(0, _RN, zrow, 0, unroll=False)

        lo8 = (lo // 8) * 8
        nch = (hi - lo8 + _CHS - 1) // _CHS

        def chunk(ci, c):
            base = lo8 + ci * _CHS
            pltpu.sync_copy(mepk_hbm.at[pl.ds(base, _CHS)], mbuf)
            start = jnp.maximum(lo - base, 0)
            cnt = jnp.minimum(hi - base, _CHS)

            def edge(e, c2):
                av = plsc.load_gather(mbuf, [_bcast16(e), _I16()])
                db = plsc.bitcast(
                    plsc.load_gather(mbuf, [_bcast16(e), _bcast16(8)]),
                    jnp.int32)
                row = db - nodebase
                cur = plsc.load_gather(macc, [row, _I16()])
                plsc.store_scatter(macc, [row, _I16()], cur + av, mask=m3)
                return c2
            lax.fori_loop(start, cnt, edge, 0, unroll=False)
            return c
        lax.fori_loop(0, nch, chunk, 0, unroll=False)
        pltpu.sync_copy(macc, macc_hbm.at[pl.ds(nodebase, _RN)])


def _mea_sc(mepk, offs):
    mesh = plsc.VectorSubcoreMesh(core_axis_name="c", subcore_axis_name="s")
    k = functools.partial(
        pl.kernel,
        mesh=mesh,
        out_type=jax.ShapeDtypeStruct((_NPAD, 16), jnp.float32),
        compiler_params=pltpu.CompilerParams(needs_layout_passes=False),
        scratch_types=[
            pltpu.VMEM((168,), jnp.int32),
            pltpu.VMEM((_CHS, 16), jnp.float32),
            pltpu.VMEM((_RN, 16), jnp.float32),
            pltpu.SemaphoreType.DMA,
        ],
    )(_mea_body)
    return k(mepk, offs)


def _sg_body(aepk_hbm, z_hbm, aself_hbm, offs_hbm,
             agg_hbm, aso_hbm, slo_hbm,
             offsbuf, abuf, zbuf, maxacc, sumacc, aselfbuf, asbuf, slbuf,
             acc, sem):
    w = lax.axis_index("s") * 2 + lax.axis_index("c")
    pltpu.sync_copy(offs_hbm, offsbuf)
    m8 = _I16() < 8
    zeros = jnp.zeros((16,), jnp.float32)

    for half in range(_RPW):
        r = _RPW * w + half
        nodebase = r * _RN
        lo, hi = _range_bounds(offsbuf, r)
        lo8 = (lo // 8) * 8
        nch = (hi - lo8 + _CHS - 1) // _CHS

        pltpu.sync_copy(aself_hbm.at[pl.ds(nodebase, _RN)], maxacc)
        pltpu.sync_copy(aself_hbm.at[pl.ds(nodebase, _RN)], aselfbuf)

        def zrow(n, c):
            plsc.store_scatter(sumacc, [_bcast16(n), _I16()], zeros)
            for j in range(HC // 16):
                plsc.store_scatter(acc, [_bcast16(n), _I16() + j * 16], zeros)
            return c
        lax.fori_loop(0, _RN, zrow, 0, unroll=False)

        def chunk1(ci, c):
            base = lo8 + ci * _CHS
            pltpu.sync_copy(aepk_hbm.at[pl.ds(base, _CHS)], abuf)
            start = jnp.maximum(lo - base, 0)
            cnt = jnp.minimum(hi - base, _CHS)

            def edge(e, c2):
                av = plsc.load_gather(abuf, [_bcast16(e), _I16()])
                db = plsc.bitcast(
                    plsc.load_gather(abuf, [_bcast16(e), _bcast16(8)]),
                    jnp.int32)
                row = db - nodebase
                cur = plsc.load_gather(maxacc, [row, _I16()])
                plsc.store_scatter(maxacc, [row, _I16()],
                                   jnp.maximum(cur, av), mask=m8)
                return c2
            lax.fori_loop(start, cnt, edge, 0, unroll=False)
            return c
        lax.fori_loop(0, nch, chunk1, 0, unroll=False)

        def chunk2(ci, c):
            base = lo8 + ci * _CHS
            pltpu.sync_copy(aepk_hbm.at[pl.ds(base, _CHS)], abuf)
            start = jnp.maximum(lo - base, 0)
            cnt = jnp.minimum(hi - base, _CHS)

            def edge(e, c2):
                av = plsc.load_gather(abuf, [_bcast16(e), _I16()])
                db = plsc.bitcast(
                    plsc.load_gather(abuf, [_bcast16(e), _bcast16(8)]),
                    jnp.int32)
                row = db - nodebase
                mx = plsc.load_gather(maxacc, [row, _I16()])
                pv = jnp.exp(av - mx)
                cur = plsc.load_gather(sumacc, [row, _I16()])
                plsc.store_scatter(sumacc, [row, _I16()], cur + pv, mask=m8)
                return c2
            lax.fori_loop(start, cnt, edge, 0, unroll=False)
            return c
        lax.fori_loop(0, nch, chunk2, 0, unroll=False)

        def node(n, c):
            nb = _bcast16(n)
            aself = plsc.load_gather(aselfbuf, [nb, _I16()])
            mx = plsc.load_gather(maxacc, [nb, _I16()])
            sm = plsc.load_gather(sumacc, [nb, _I16()])
            pself = jnp.exp(aself - mx)
            inv = 1.0 / (sm + pself + 1e-16)
            plsc.store_scatter(sumacc, [nb, _I16()], inv, mask=m8)
            plsc.store_scatter(asbuf, [nb, _I16()], pself * inv)
            plsc.store_scatter(slbuf, [nb, _I16()], sm * inv)
            return c
        lax.fori_loop(0, _RN, node, 0, unroll=False)

        def chunk3(ci, c):
            base = lo8 + ci * _CHS
            pltpu.sync_copy(aepk_hbm.at[pl.ds(base, _CHS)], abuf)
            pltpu.sync_copy(z_hbm.at[pl.ds(base, _CHS)], zbuf)
            start = jnp.maximum(lo - base, 0)
            cnt = jnp.minimum(hi - base, _CHS)

            def edge(e, c2):
                eb = _bcast16(e)
                db = plsc.bitcast(
                    plsc.load_gather(abuf, [eb, _bcast16(8)]), jnp.int32)
                row = db - nodebase
                ws = []
                for h in range(H):
                    hv = _bcast16(h)
                    avh = plsc.load_gather(abuf, [eb, hv])
                    mh = plsc.load_gather(maxacc, [row, hv])
                    ivh = plsc.load_gather(sumacc, [row, hv])
                    ws.append(jnp.exp(avh - mh) * ivh)
                for j in range(HC // 16):
                    col = _I16() + j * 16
                    zv = plsc.load_gather(zbuf, [eb, col])
                    plsc.addupdate_scatter(acc, [row, col], zv * ws[j // 4])
                return c2
            lax.fori_loop(start, cnt, edge, 0, unroll=False)
            return c
        lax.fori_loop(0, nch, chunk3, 0, unroll=False)

        pltpu.sync_copy(acc, agg_hbm.at[pl.ds(nodebase, _RN)])
        pltpu.sync_copy(asbuf, aso_hbm.at[pl.ds(nodebase, _RN)])
        pltpu.sync_copy(slbuf, slo_hbm.at[pl.ds(nodebase, _RN)])


def _sg_sc(aepk, zsum, aselfpk, offs):
    mesh = plsc.VectorSubcoreMesh(core_axis_name="c", subcore_axis_name="s")
    k = functools.partial(
        pl.kernel,
        mesh=mesh,
        compiler_params=pltpu.CompilerParams(needs_layout_passes=False),
        out_type=(jax.ShapeDtypeStruct((_NPAD, HC), jnp.float32),
                  jax.ShapeDtypeStruct((_NPAD, 16), jnp.float32),
                  jax.ShapeDtypeStruct((_NPAD, 16), jnp.float32)),
        scratch_types=[
            pltpu.VMEM((168,), jnp.int32),
            pltpu.VMEM((_CHS, 16), jnp.float32),
            pltpu.VMEM((_CHS, HC), jnp.float32),
            pltpu.VMEM((_RN, 16), jnp.float32),
            pltpu.VMEM((_RN, 16), jnp.float32),
            pltpu.VMEM((_RN, 16), jnp.float32),
            pltpu.VMEM((_RN, 16), jnp.float32),
            pltpu.VMEM((_RN, 16), jnp.float32),
            pltpu.VMEM((_RN, HC), jnp.float32),
            pltpu.SemaphoreType.DMA,
        ],
    )(_sg_body)
    return k(aepk, zsum, aselfpk, offs)


_EB = 2000


def _escore_body(z_ref, ea_ref, we_ref, attf_ref, sel_ref, m_ref, d_ref,
                 o_ref):
    ee = lax.dot_general(ea_ref[...], we_ref[...], (((1,), (1,)), ((), ())),
                         preferred_element_type=jnp.float32)
    z = z_ref[...] + ee
    z = jnp.where(z >= 0.0, z, 0.2 * z)
    a = lax.dot_general(z * attf_ref[...], sel_ref[...],
                        (((1,), (0,)), ((), ())),
                        preferred_element_type=jnp.float32)
    a = jnp.where(m_ref[...] > 0.0, a, -jnp.inf)
    zpad = jnp.zeros((a.shape[0], 7), jnp.float32)
    o_ref[...] = jnp.concatenate([a, d_ref[...], zpad], axis=1)


def _edge_scores(zsum, sea, We, att, maskf, sdstf):
    attf = att.reshape(1, HC)
    sel = jnp.zeros((HC, H), jnp.float32)
    sel = sel.at[jnp.arange(HC), jnp.arange(HC) // C].set(1.0)
    nb = E // _EB
    return pl.pallas_call(
        _escore_body,
        grid=(nb,),
        in_specs=[
            pl.BlockSpec((_EB, HC), lambda i: (i, 0)),
            pl.BlockSpec((_EB, 2), lambda i: (i, 0)),
            pl.BlockSpec((HC, 2), lambda i: (0, 0)),
            pl.BlockSpec((1, HC), lambda i: (0, 0)),
            pl.BlockSpec((HC, H), lambda i: (0, 0)),
            pl.BlockSpec((_EB, 1), lambda i: (i, 0)),
            pl.BlockSpec((_EB, 1), lambda i: (i, 0)),
        ],
        out_specs=pl.BlockSpec((_EB, 16), lambda i: (i, 0)),
        out_shape=jax.ShapeDtypeStruct((E, 16), jnp.float32),
    )(zsum, sea, We, attf, sel, maskf.reshape(E, 1), sdstf.reshape(E, 1))


def _combine_body(agg_ref, xl_ref, xr_ref, as_ref, sl_ref, b_ref, exp_ref,
                  o_ref):
    aexp = lax.dot_general(as_ref[...], exp_ref[...], (((1,), (0,)), ((), ())),
                           preferred_element_type=jnp.float32)
    sexp = lax.dot_general(sl_ref[...], exp_ref[...], (((1,), (0,)), ((), ())),
                           preferred_element_type=jnp.float32)
    out = (agg_ref[...] - sexp * xr_ref[...] + aexp * xl_ref[...]
           + b_ref[...])
    o_ref[...] = jnp.maximum(out, 0.0)


def _combine(agg, xl, xr, aself_o, salpha_o, bias):
    exp16 = jnp.zeros((16, HC), jnp.float32)
    exp16 = exp16.at[jnp.arange(HC) // C, jnp.arange(HC)].set(1.0)
    nb = N // _RB
    return pl.pallas_call(
        _combine_body,
        grid=(nb,),
        in_specs=[
            pl.BlockSpec((_RB, HC), lambda i: (i, 0)),
            pl.BlockSpec((_RB, HC), lambda i: (i, 0)),
            pl.BlockSpec((_RB, HC), lambda i: (i, 0)),
            pl.BlockSpec((_RB, 16), lambda i: (i, 0)),
            pl.BlockSpec((_RB, 16), lambda i: (i, 0)),
            pl.BlockSpec((1, HC), lambda i: (0, 0)),
            pl.BlockSpec((16, HC), lambda i: (0, 0)),
        ],
        out_specs=pl.BlockSpec((_RB, HC), lambda i: (i, 0)),
        out_shape=jax.ShapeDtypeStruct((N, HC), jnp.float32),
    )(agg, xl, xr, aself_o, salpha_o, bias.reshape(1, HC), exp16)


def _gat_layer(h, ssrc, sdst, sdstf, sea, offs, p, smaskf):
    xl, xr = _gat_proj(h, p)

    mea = sea * smaskf[:, None]
    mepk = jnp.concatenate(
        [mea, smaskf[:, None], jnp.zeros((E, 5), jnp.float32),
         sdstf[:, None], jnp.zeros((E, 7), jnp.float32)], axis=1)
    mepk = jnp.concatenate([mepk, jnp.zeros((64, 16), jnp.float32)], axis=0)
    macc = _mea_sc(mepk, offs)
    mean_ea = macc[:N, 0:2] / jnp.maximum(macc[:N, 2:3], 1.0)

    a_self = _self_scores(xl, xr, mean_ea, p["We"], p["att"])
    aselfpk = jnp.pad(a_self, ((0, _NPAD - N), (0, 8)))

    zsum = _zsum_sc(xl, xr, ssrc, sdst)
    a16 = _edge_scores(zsum, sea, p["We"], p["att"], smaskf, sdstf)
    aepk = jnp.concatenate([a16, jnp.zeros((64, 16), jnp.float32)], axis=0)

    agg, aself_o, salpha_o = _sg_sc(aepk, zsum, aselfpk, offs)

    return _combine(agg, xl, xr, aself_o, salpha_o, p["bias"])


def kernel(x, edge_index, edge_attr, params):
    src = edge_index[0]
    dst = edge_index[1]
    a0 = edge_attr[:, 0]
    fea = edge_attr[:, 1:]
    p = params

    perm = jnp.argsort(dst)
    ssrc = src[perm]
    sdst = dst[perm]
    sea = fea[perm]
    sa0 = a0[perm]
    sdstf = lax.bitcast_convert_type(sdst, jnp.float32)
    bounds = jnp.minimum(jnp.arange(_NR + 1, dtype=jnp.int32) * _RN, N)
    offs = jnp.searchsorted(sdst, bounds, side="left").astype(jnp.int32)
    offs = jnp.concatenate([offs, jnp.zeros((168 - _NR - 1,), jnp.int32)])
    m1 = (sa0 >= 0).astype(jnp.float32)
    m2 = (sa0 <= 0).astype(jnp.float32)
    mall = jnp.ones((E,), jnp.float32)

    h = _lin_bn_relu(x, p["W0"], p["b0"], p["g0"], p["be0"])
    h = _gat_layer(h, ssrc, sdst, sdstf, sea, offs, p["gat1"], m1)
    h = _lin_bn_relu(h, p["W1"], p["b1"], p["g1"], p["be1"])
    h = _gat_layer(h, ssrc, sdst, sdstf, sea, offs, p["gat2"], m2)
    h = _lin_bn_relu(h, p["W2"], p["b2"], p["g2"], p["be2"])
    h = _gat_layer(h, ssrc, sdst, sdstf, sea, offs, p["gat3"], mall)
    return _final_stage(h, p["W3"], p["b3"], p["g3"], p["be3"],
                        p["WF"], p["bF"])


# speedup vs baseline: 7.8009x; 1.0603x over previous
import functools

import jax
import jax.numpy as jnp
from jax import lax
from jax.experimental import pallas as pl
from jax.experimental.pallas import tpu as pltpu
from jax.experimental.pallas import tpu_sc as plsc

N = 10000
E = 160000
FEAT = 256
C = 64
H = 8
HC = H * C


def _linbn_body(h_ref, w_ref, b_ref, g_ref, be_ref, o_ref):
    hh = lax.dot_general(h_ref[...], w_ref[...],
                         (((1,), (1,)), ((), ())),
                         preferred_element_type=jnp.float32)
    hh = hh + b_ref[...]
    m = jnp.mean(hh, axis=0, keepdims=True)
    v = jnp.mean((hh - m) ** 2, axis=0, keepdims=True)
    o_ref[...] = jnp.maximum((hh - m) / jnp.sqrt(v + 1e-5) * g_ref[...]
                             + be_ref[...], 0.0)


def _lin_bn_relu(h, W, b, g, be):
    n, _ = h.shape
    co = W.shape[0]
    return pl.pallas_call(
        _linbn_body,
        out_shape=jax.ShapeDtypeStruct((n, co), jnp.float32),
    )(h, W, b.reshape(1, co), g.reshape(1, co), be.reshape(1, co))


def _proj_body(h_ref, wl_ref, bl_ref, wr_ref, br_ref, xl_ref, xr_ref):
    h = h_ref[...]
    xl_ref[...] = lax.dot_general(h, wl_ref[...], (((1,), (1,)), ((), ())),
                                  preferred_element_type=jnp.float32) + bl_ref[...]
    xr_ref[...] = lax.dot_general(h, wr_ref[...], (((1,), (1,)), ((), ())),
                                  preferred_element_type=jnp.float32) + br_ref[...]


_RB = 2000


def _gat_proj(h, p):
    n = h.shape[0]
    nb = n // _RB
    return pl.pallas_call(
        _proj_body,
        grid=(nb,),
        in_specs=[
            pl.BlockSpec((_RB, C), lambda i: (i, 0)),
            pl.BlockSpec((HC, C), lambda i: (0, 0)),
            pl.BlockSpec((1, HC), lambda i: (0, 0)),
            pl.BlockSpec((HC, C), lambda i: (0, 0)),
            pl.BlockSpec((1, HC), lambda i: (0, 0)),
        ],
        out_specs=(pl.BlockSpec((_RB, HC), lambda i: (i, 0)),
                   pl.BlockSpec((_RB, HC), lambda i: (i, 0))),
        out_shape=(jax.ShapeDtypeStruct((n, HC), jnp.float32),
                   jax.ShapeDtypeStruct((n, HC), jnp.float32)),
    )(h, p["Wl"], p["bl"].reshape(1, HC), p["Wr"], p["br"].reshape(1, HC))


def _self_score_body(xl_ref, xr_ref, mea_ref, we_ref, attf_ref, sel_ref,
                     a_ref):
    ee = lax.dot_general(mea_ref[...], we_ref[...], (((1,), (1,)), ((), ())),
                         preferred_element_type=jnp.float32)
    z = xl_ref[...] + xr_ref[...] + ee
    z = jnp.where(z >= 0.0, z, 0.2 * z)
    za = z * attf_ref[...]
    a_ref[...] = lax.dot_general(za, sel_ref[...], (((1,), (0,)), ((), ())),
                                 preferred_element_type=jnp.float32)


def _self_scores(xl, xr, mean_ea, We, att):
    attf = att.reshape(1, HC)
    sel = jnp.zeros((HC, H), jnp.float32)
    sel = sel.at[jnp.arange(HC), jnp.arange(HC) // C].set(1.0)
    nb = N // _RB
    return pl.pallas_call(
        _self_score_body,
        grid=(nb,),
        in_specs=[
            pl.BlockSpec((_RB, HC), lambda i: (i, 0)),
            pl.BlockSpec((_RB, HC), lambda i: (i, 0)),
            pl.BlockSpec((_RB, 2), lambda i: (i, 0)),
            pl.BlockSpec((HC, 2), lambda i: (0, 0)),
            pl.BlockSpec((1, HC), lambda i: (0, 0)),
            pl.BlockSpec((HC, H), lambda i: (0, 0)),
        ],
        out_specs=pl.BlockSpec((_RB, H), lambda i: (i, 0)),
        out_shape=jax.ShapeDtypeStruct((N, H), jnp.float32),
    )(xl, xr, mean_ea, We, attf, sel)


def _final_body(h_ref, w_ref, b_ref, g_ref, be_ref, wf_ref, bf_ref, o_ref):
    hh = lax.dot_general(h_ref[...], w_ref[...], (((1,), (1,)), ((), ())),
                         preferred_element_type=jnp.float32) + b_ref[...]
    m = jnp.mean(hh, axis=0, keepdims=True)
    v = jnp.mean((hh - m) ** 2, axis=0, keepdims=True)
    hh = jnp.maximum((hh - m) / jnp.sqrt(v + 1e-5) * g_ref[...] + be_ref[...],
                     0.0)
    lg = lax.dot_general(hh, wf_ref[...], (((1,), (1,)), ((), ())),
                         preferred_element_type=jnp.float32) + bf_ref[...]
    mx = jnp.max(lg, axis=1, keepdims=True)
    el = jnp.exp(lg - mx)
    sm = el / jnp.sum(el, axis=1, keepdims=True)
    o_ref[...] = sm[:, 1:]


def _final_stage(h, W, b, g, be, WF, bF):
    return pl.pallas_call(
        _final_body,
        out_shape=jax.ShapeDtypeStruct((N, 1), jnp.float32),
    )(h, W, b.reshape(1, C), g.reshape(1, C), be.reshape(1, C),
      WF, bF.reshape(1, 2))


_NW = 32
_EPW = E // _NW
_CH = 40
_NCHUNK = _EPW // _CH


def _zsum_body(xl_hbm, xr_hbm, src_hbm, dst_hbm, z_hbm,
               sidx, didx, xlb, xrb, sem1, sem2):
    w = lax.axis_index("s") * 2 + lax.axis_index("c")
    base = w * _EPW
    pltpu.sync_copy(src_hbm.at[pl.ds(base, _EPW)], sidx)
    pltpu.sync_copy(dst_hbm.at[pl.ds(base, _EPW)], didx)

    def chunk(ci, carry):
        o = ci * _CH
        cp1 = pltpu.async_copy(xl_hbm.at[sidx.at[pl.ds(o, _CH)]], xlb, sem1)
        cp2 = pltpu.async_copy(xr_hbm.at[didx.at[pl.ds(o, _CH)]], xrb, sem2)
        cp1.wait()
        cp2.wait()

        def row(r, c2):
            for j in range(HC // 16):
                xlb[r, pl.ds(j * 16, 16)] = (xlb[r, pl.ds(j * 16, 16)]
                                             + xrb[r, pl.ds(j * 16, 16)])
            return c2

        lax.fori_loop(0, _CH, row, 0, unroll=False)
        pltpu.sync_copy(xlb, z_hbm.at[pl.ds(base + o, _CH)])
        return carry

    lax.fori_loop(0, _NCHUNK, chunk, 0, unroll=False)


_EPAD = E + 128


def _zsum_sc(xl, xr, ssrc, sdst, offs):
    del offs
    mesh = plsc.VectorSubcoreMesh(core_axis_name="c", subcore_axis_name="s")
    k = functools.partial(
        pl.kernel,
        mesh=mesh,
        compiler_params=pltpu.CompilerParams(needs_layout_passes=False),
        out_type=jax.ShapeDtypeStruct((_EPAD, HC), jnp.float32),
        scratch_types=[
            pltpu.VMEM((_EPW,), jnp.int32),
            pltpu.VMEM((_EPW,), jnp.int32),
            pltpu.VMEM((_CH, HC), jnp.float32),
            pltpu.VMEM((_CH, HC), jnp.float32),
            pltpu.SemaphoreType.DMA,
            pltpu.SemaphoreType.DMA,
        ],
    )(_zsum_body)
    return k(xl, xr, ssrc, sdst)


_NR = 160
_RN = 64
_RPW = _NR // _NW
_CHS = 24
_CHA = 96
_NPAD = _NR * _RN
_I16 = lambda: lax.iota(jnp.int32, 16)


def _bcast16(v):
    return jnp.full((16,), v, jnp.int32)


def _scalar(ref, i):
    v = plsc.load_gather(ref, [_bcast16(i)])
    return lax.reduce_max(v, axes=(0,))


def _range_bounds(offsbuf, r):
    lo = _scalar(offsbuf, r)
    hi = _scalar(offsbuf, r + 1)
    return lo, hi


def _mea_body(mepk_hbm, offs_hbm, macc_hbm, offsbuf, mbuf, macc, sem):
    w = lax.axis_index("s") * 2 + lax.axis_index("c")
    pltpu.sync_copy(offs_hbm, offsbuf)
    m3 = _I16() < 3
    zeros = jnp.zeros((16,), jnp.float32)
    for half in range(_RPW):
        r = _RPW * w + half
        nodebase = r * _RN
        lo, hi = _range_bounds(offsbuf, r)

        def zrow(n, c):
            plsc.store_scatter(macc, [_bcast16(n), _I16()], zeros)
            return c
        lax.fori_loop(0, _RN, zrow, 0, unroll=False)

        lo8 = (lo // 8) * 8
        nch = (hi - lo8 + _CHS - 1) // _CHS

        def chunk(ci, c):
            base = lo8 + ci * _CHS
            pltpu.sync_copy(mepk_hbm.at[pl.ds(base, _CHS)], mbuf)
            start = jnp.maximum(lo - base, 0)
            cnt = jnp.minimum(hi - base, _CHS)

            def edge(e, c2):
                av = plsc.load_gather(mbuf, [_bcast16(e), _I16()])
                db = plsc.bitcast(
                    plsc.load_gather(mbuf, [_bcast16(e), _bcast16(8)]),
                    jnp.int32)
                row = db - nodebase
                cur = plsc.load_gather(macc, [row, _I16()])
                plsc.store_scatter(macc, [row, _I16()], cur + av, mask=m3)
                return c2
            lax.fori_loop(start, cnt, edge, 0, unroll=False)
            return c
        lax.fori_loop(0, nch, chunk, 0, unroll=False)
        pltpu.sync_copy(macc, macc_hbm.at[pl.ds(nodebase, _RN)])


def _mea_sc(mepk, offs):
    mesh = plsc.VectorSubcoreMesh(core_axis_name="c", subcore_axis_name="s")
    k = functools.partial(
        pl.kernel,
        mesh=mesh,
        out_type=jax.ShapeDtypeStruct((_NPAD, 16), jnp.float32),
        compiler_params=pltpu.CompilerParams(needs_layout_passes=False),
        scratch_types=[
            pltpu.VMEM((168,), jnp.int32),
            pltpu.VMEM((_CHS, 16), jnp.float32),
            pltpu.VMEM((_RN, 16), jnp.float32),
            pltpu.SemaphoreType.DMA,
        ],
    )(_mea_body)
    return k(mepk, offs)


def _sg_body(aepk_hbm, z_hbm, aself_hbm, offs_hbm,
             agg_hbm, aso_hbm, slo_hbm,
             offsbuf, abuf, zbuf, maxacc, sumacc, aselfbuf, asbuf, slbuf,
             acc, wbuf, sem):
    w = lax.axis_index("s") * 2 + lax.axis_index("c")
    pltpu.sync_copy(offs_hbm, offsbuf)
    m8 = _I16() < 8
    zeros = jnp.zeros((16,), jnp.float32)

    for half in range(_RPW):
        r = _RPW * w + half
        nodebase = r * _RN
        lo, hi = _range_bounds(offsbuf, r)
        lo8 = (lo // 8) * 8
        nch = (hi - lo8 + _CHS - 1) // _CHS
        ncha = (hi - lo8 + _CHA - 1) // _CHA

        pltpu.sync_copy(aself_hbm.at[pl.ds(nodebase, _RN)], maxacc)
        pltpu.sync_copy(aself_hbm.at[pl.ds(nodebase, _RN)], aselfbuf)

        def zrow(n, c):
            plsc.store_scatter(sumacc, [_bcast16(n), _I16()], zeros)
            for j in range(HC // 16):
                plsc.store_scatter(acc, [_bcast16(n), _I16() + j * 16], zeros)
            return c
        lax.fori_loop(0, _RN, zrow, 0, unroll=False)

        def chunk1(ci, c):
            base = lo8 + ci * _CHA
            pltpu.sync_copy(aepk_hbm.at[pl.ds(base, _CHA)], abuf)
            start = jnp.maximum(lo - base, 0)
            cnt = jnp.minimum(hi - base, _CHA)

            def edge(e, c2):
                av = plsc.load_gather(abuf, [_bcast16(e), _I16()])
                db = plsc.bitcast(
                    plsc.load_gather(abuf, [_bcast16(e), _bcast16(8)]),
                    jnp.int32)
                row = db - nodebase
                cur = plsc.load_gather(maxacc, [row, _I16()])
                plsc.store_scatter(maxacc, [row, _I16()],
                                   jnp.maximum(cur, av), mask=m8)
                return c2
            lax.fori_loop(start, cnt, edge, 0, unroll=False)
            return c
        lax.fori_loop(0, ncha, chunk1, 0, unroll=False)

        def chunk2(ci, c):
            base = lo8 + ci * _CHA
            pltpu.sync_copy(aepk_hbm.at[pl.ds(base, _CHA)], abuf)
            start = jnp.maximum(lo - base, 0)
            cnt = jnp.minimum(hi - base, _CHA)

            def edge(e, c2):
                av = plsc.load_gather(abuf, [_bcast16(e), _I16()])
                db = plsc.bitcast(
                    plsc.load_gather(abuf, [_bcast16(e), _bcast16(8)]),
                    jnp.int32)
                row = db - nodebase
                mx = plsc.load_gather(maxacc, [row, _I16()])
                pv = jnp.exp(av - mx)
                cur = plsc.load_gather(sumacc, [row, _I16()])
                plsc.store_scatter(sumacc, [row, _I16()], cur + pv, mask=m8)
                return c2
            lax.fori_loop(start, cnt, edge, 0, unroll=False)
            return c
        lax.fori_loop(0, ncha, chunk2, 0, unroll=False)

        def node(n, c):
            nb = _bcast16(n)
            aself = plsc.load_gather(aselfbuf, [nb, _I16()])
            mx = plsc.load_gather(maxacc, [nb, _I16()])
            sm = plsc.load_gather(sumacc, [nb, _I16()])
            pself = jnp.exp(aself - mx)
            inv = 1.0 / (sm + pself + 1e-16)
            plsc.store_scatter(sumacc, [nb, _I16()], inv, mask=m8)
            plsc.store_scatter(asbuf, [nb, _I16()], pself * inv)
            plsc.store_scatter(slbuf, [nb, _I16()], sm * inv)
            return c
        lax.fori_loop(0, _RN, node, 0, unroll=False)

        def chunk3(ci, c):
            base = lo8 + ci * _CHS
            pltpu.sync_copy(aepk_hbm.at[pl.ds(base, _CHS)], abuf.at[pl.ds(0, _CHS)])
            pltpu.sync_copy(z_hbm.at[pl.ds(base, _CHS)], zbuf)
            start = jnp.maximum(lo - base, 0)
            cnt = jnp.minimum(hi - base, _CHS)

            def edge(e, c2):
                eb = _bcast16(e)
                db = plsc.bitcast(
                    plsc.load_gather(abuf, [eb, _bcast16(8)]), jnp.int32)
                row = db - nodebase
                ws = []
                for h in range(H):
                    hv = _bcast16(h)
                    avh = plsc.load_gather(abuf, [eb, hv])
                    mh = plsc.load_gather(maxacc, [row, hv])
                    ivh = plsc.load_gather(sumacc, [row, hv])
                    ws.append(jnp.exp(avh - mh) * ivh)
                for j in range(HC // 16):
                    col = _I16() + j * 16
                    zv = plsc.load_gather(zbuf, [eb, col])
                    plsc.addupdate_scatter(acc, [row, col], zv * ws[j // 4])
                return c2
            lax.fori_loop(start, cnt, edge, 0, unroll=False)
            return c
        lax.fori_loop(0, nch, chunk3, 0, unroll=False)

        pltpu.sync_copy(acc, agg_hbm.at[pl.ds(nodebase, _RN)])
        pltpu.sync_copy(asbuf, aso_hbm.at[pl.ds(nodebase, _RN)])
        pltpu.sync_copy(slbuf, slo_hbm.at[pl.ds(nodebase, _RN)])


def _sg_sc(aepk, zsum, aselfpk, offs):
    mesh = plsc.VectorSubcoreMesh(core_axis_name="c", subcore_axis_name="s")
    k = functools.partial(
        pl.kernel,
        mesh=mesh,
        compiler_params=pltpu.CompilerParams(needs_layout_passes=False),
        out_type=(jax.ShapeDtypeStruct((_NPAD, HC), jnp.float32),
                  jax.ShapeDtypeStruct((_NPAD, 16), jnp.float32),
                  jax.ShapeDtypeStruct((_NPAD, 16), jnp.float32)),
        scratch_types=[
            pltpu.VMEM((168,), jnp.int32),
            pltpu.VMEM((_CHA, 16), jnp.float32),
            pltpu.VMEM((_CHS, HC), jnp.float32),
            pltpu.VMEM((_RN, 16), jnp.float32),
            pltpu.VMEM((_RN, 16), jnp.float32),
            pltpu.VMEM((_RN, 16), jnp.float32),
            pltpu.VMEM((_RN, 16), jnp.float32),
            pltpu.VMEM((_RN, 16), jnp.float32),
            pltpu.VMEM((_RN, HC), jnp.float32),
            pltpu.VMEM((16,), jnp.float32),
            pltpu.SemaphoreType.DMA,
        ],
    )(_sg_body)
    return k(aepk, zsum, aselfpk, offs)


_EB = 2000


def _escore_body(z_ref, ea_ref, we_ref, attf_ref, sel_ref, m_ref, d_ref,
                 o_ref):
    ee = lax.dot_general(ea_ref[...], we_ref[...], (((1,), (1,)), ((), ())),
                         preferred_element_type=jnp.float32)
    z = z_ref[...] + ee
    z = jnp.where(z >= 0.0, z, 0.2 * z)
    a = lax.dot_general(z * attf_ref[...], sel_ref[...],
                        (((1,), (0,)), ((), ())),
                        preferred_element_type=jnp.float32)
    a = jnp.where(m_ref[...] > 0.0, a, -jnp.inf)
    zpad = jnp.zeros((a.shape[0], 7), jnp.float32)
    o_ref[...] = jnp.concatenate([a, d_ref[...], zpad], axis=1)


def _edge_scores(zsum, sea, We, att, maskf, sdstf):
    attf = att.reshape(1, HC)
    sel = jnp.zeros((HC, H), jnp.float32)
    sel = sel.at[jnp.arange(HC), jnp.arange(HC) // C].set(1.0)
    nb = E // _EB
    return pl.pallas_call(
        _escore_body,
        grid=(nb,),
        in_specs=[
            pl.BlockSpec((_EB, HC), lambda i: (i, 0)),
            pl.BlockSpec((_EB, 2), lambda i: (i, 0)),
            pl.BlockSpec((HC, 2), lambda i: (0, 0)),
            pl.BlockSpec((1, HC), lambda i: (0, 0)),
            pl.BlockSpec((HC, H), lambda i: (0, 0)),
            pl.BlockSpec((_EB, 1), lambda i: (i, 0)),
            pl.BlockSpec((_EB, 1), lambda i: (i, 0)),
        ],
        out_specs=pl.BlockSpec((_EB, 16), lambda i: (i, 0)),
        out_shape=jax.ShapeDtypeStruct((E, 16), jnp.float32),
    )(zsum, sea, We, attf, sel, maskf.reshape(E, 1), sdstf.reshape(E, 1))


def _combine_body(agg_ref, xl_ref, xr_ref, as_ref, sl_ref, b_ref, exp_ref,
                  o_ref):
    aexp = lax.dot_general(as_ref[...], exp_ref[...], (((1,), (0,)), ((), ())),
                           preferred_element_type=jnp.float32)
    sexp = lax.dot_general(sl_ref[...], exp_ref[...], (((1,), (0,)), ((), ())),
                           preferred_element_type=jnp.float32)
    out = (agg_ref[...] - sexp * xr_ref[...] + aexp * xl_ref[...]
           + b_ref[...])
    o_ref[...] = jnp.maximum(out, 0.0)


def _combine(agg, xl, xr, aself_o, salpha_o, bias):
    exp16 = jnp.zeros((16, HC), jnp.float32)
    exp16 = exp16.at[jnp.arange(HC) // C, jnp.arange(HC)].set(1.0)
    nb = N // _RB
    return pl.pallas_call(
        _combine_body,
        grid=(nb,),
        in_specs=[
            pl.BlockSpec((_RB, HC), lambda i: (i, 0)),
            pl.BlockSpec((_RB, HC), lambda i: (i, 0)),
            pl.BlockSpec((_RB, HC), lambda i: (i, 0)),
            pl.BlockSpec((_RB, 16), lambda i: (i, 0)),
            pl.BlockSpec((_RB, 16), lambda i: (i, 0)),
            pl.BlockSpec((1, HC), lambda i: (0, 0)),
            pl.BlockSpec((16, HC), lambda i: (0, 0)),
        ],
        out_specs=pl.BlockSpec((_RB, HC), lambda i: (i, 0)),
        out_shape=jax.ShapeDtypeStruct((N, HC), jnp.float32),
    )(agg, xl, xr, aself_o, salpha_o, bias.reshape(1, HC), exp16)


def _gat_layer(h, ssrc, sdst, sdstf, sea, offs, p, smaskf):
    xl, xr = _gat_proj(h, p)

    mea = sea * smaskf[:, None]
    mepk = jnp.concatenate(
        [mea, smaskf[:, None], jnp.zeros((E, 5), jnp.float32),
         sdstf[:, None], jnp.zeros((E, 7), jnp.float32)], axis=1)
    mepk = jnp.concatenate([mepk, jnp.zeros((128, 16), jnp.float32)], axis=0)
    macc = _mea_sc(mepk, offs)
    mean_ea = macc[:N, 0:2] / jnp.maximum(macc[:N, 2:3], 1.0)

    a_self = _self_scores(xl, xr, mean_ea, p["We"], p["att"])
    aselfpk = jnp.pad(a_self, ((0, _NPAD - N), (0, 8)))

    zsum = _zsum_sc(xl, xr, ssrc, sdst, offs)
    a16 = _edge_scores(zsum, sea, p["We"], p["att"], smaskf, sdstf)
    aepk = jnp.concatenate([a16, jnp.zeros((128, 16), jnp.float32)], axis=0)

    agg, aself_o, salpha_o = _sg_sc(aepk, zsum, aselfpk, offs)

    return _combine(agg, xl, xr, aself_o, salpha_o, p["bias"])


def kernel(x, edge_index, edge_attr, params):
    src = edge_index[0]
    dst = edge_index[1]
    a0 = edge_attr[:, 0]
    fea = edge_attr[:, 1:]
    p = params

    perm = jnp.argsort(dst)
    ssrc = src[perm]
    sdst = dst[perm]
    sea = fea[perm]
    sa0 = a0[perm]
    sdstf = lax.bitcast_convert_type(sdst, jnp.float32)
    bounds = jnp.minimum(jnp.arange(_NR + 1, dtype=jnp.int32) * _RN, N)
    offs = jnp.searchsorted(sdst, bounds, side="left").astype(jnp.int32)
    offs = jnp.concatenate([offs, jnp.zeros((168 - _NR - 1,), jnp.int32)])
    m1 = (sa0 >= 0).astype(jnp.float32)
    m2 = (sa0 <= 0).astype(jnp.float32)
    mall = jnp.ones((E,), jnp.float32)

    h = _lin_bn_relu(x, p["W0"], p["b0"], p["g0"], p["be0"])
    h = _gat_layer(h, ssrc, sdst, sdstf, sea, offs, p["gat1"], m1)
    h = _lin_bn_relu(h, p["W1"], p["b1"], p["g1"], p["be1"])
    h = _gat_layer(h, ssrc, sdst, sdstf, sea, offs, p["gat2"], m2)
    h = _lin_bn_relu(h, p["W2"], p["b2"], p["g2"], p["be2"])
    h = _gat_layer(h, ssrc, sdst, sdstf, sea, offs, p["gat3"], mall)
    return _final_stage(h, p["W3"], p["b3"], p["g3"], p["be3"],
                        p["WF"], p["bF"])


# speedup vs baseline: 8.5451x; 1.0954x over previous
import functools

import jax
import jax.numpy as jnp
from jax import lax
from jax.experimental import pallas as pl
from jax.experimental.pallas import tpu as pltpu
from jax.experimental.pallas import tpu_sc as plsc

N = 10000
E = 160000
FEAT = 256
C = 64
H = 8
HC = H * C


def _linbn_body(h_ref, w_ref, b_ref, g_ref, be_ref, o_ref):
    hh = lax.dot_general(h_ref[...], w_ref[...],
                         (((1,), (1,)), ((), ())),
                         preferred_element_type=jnp.float32)
    hh = hh + b_ref[...]
    m = jnp.mean(hh, axis=0, keepdims=True)
    v = jnp.mean((hh - m) ** 2, axis=0, keepdims=True)
    o_ref[...] = jnp.maximum((hh - m) / jnp.sqrt(v + 1e-5) * g_ref[...]
                             + be_ref[...], 0.0)


def _lin_bn_relu(h, W, b, g, be):
    n, _ = h.shape
    co = W.shape[0]
    return pl.pallas_call(
        _linbn_body,
        out_shape=jax.ShapeDtypeStruct((n, co), jnp.float32),
    )(h, W, b.reshape(1, co), g.reshape(1, co), be.reshape(1, co))


def _proj_body(h_ref, wl_ref, bl_ref, wr_ref, br_ref, xl_ref, xr_ref):
    h = h_ref[...]
    xl_ref[...] = lax.dot_general(h, wl_ref[...], (((1,), (1,)), ((), ())),
                                  preferred_element_type=jnp.float32) + bl_ref[...]
    xr_ref[...] = lax.dot_general(h, wr_ref[...], (((1,), (1,)), ((), ())),
                                  preferred_element_type=jnp.float32) + br_ref[...]


_RB = 2000


def _gat_proj(h, p):
    n = h.shape[0]
    nb = n // _RB
    return pl.pallas_call(
        _proj_body,
        grid=(nb,),
        in_specs=[
            pl.BlockSpec((_RB, C), lambda i: (i, 0)),
            pl.BlockSpec((HC, C), lambda i: (0, 0)),
            pl.BlockSpec((1, HC), lambda i: (0, 0)),
            pl.BlockSpec((HC, C), lambda i: (0, 0)),
            pl.BlockSpec((1, HC), lambda i: (0, 0)),
        ],
        out_specs=(pl.BlockSpec((_RB, HC), lambda i: (i, 0)),
                   pl.BlockSpec((_RB, HC), lambda i: (i, 0))),
        out_shape=(jax.ShapeDtypeStruct((n, HC), jnp.float32),
                   jax.ShapeDtypeStruct((n, HC), jnp.float32)),
    )(h, p["Wl"], p["bl"].reshape(1, HC), p["Wr"], p["br"].reshape(1, HC))


def _self_score_body(xl_ref, xr_ref, mea_ref, we_ref, attf_ref, sel_ref,
                     a_ref):
    ee = lax.dot_general(mea_ref[...], we_ref[...], (((1,), (1,)), ((), ())),
                         preferred_element_type=jnp.float32)
    z = xl_ref[...] + xr_ref[...] + ee
    z = jnp.where(z >= 0.0, z, 0.2 * z)
    za = z * attf_ref[...]
    a_ref[...] = lax.dot_general(za, sel_ref[...], (((1,), (0,)), ((), ())),
                                 preferred_element_type=jnp.float32)


def _self_scores(xl, xr, mean_ea, We, att):
    attf = att.reshape(1, HC)
    sel = jnp.zeros((HC, H), jnp.float32)
    sel = sel.at[jnp.arange(HC), jnp.arange(HC) // C].set(1.0)
    nb = N // _RB
    return pl.pallas_call(
        _self_score_body,
        grid=(nb,),
        in_specs=[
            pl.BlockSpec((_RB, HC), lambda i: (i, 0)),
            pl.BlockSpec((_RB, HC), lambda i: (i, 0)),
            pl.BlockSpec((_RB, 2), lambda i: (i, 0)),
            pl.BlockSpec((HC, 2), lambda i: (0, 0)),
            pl.BlockSpec((1, HC), lambda i: (0, 0)),
            pl.BlockSpec((HC, H), lambda i: (0, 0)),
        ],
        out_specs=pl.BlockSpec((_RB, H), lambda i: (i, 0)),
        out_shape=jax.ShapeDtypeStruct((N, H), jnp.float32),
    )(xl, xr, mean_ea, We, attf, sel)


def _final_body(h_ref, w_ref, b_ref, g_ref, be_ref, wf_ref, bf_ref, o_ref):
    hh = lax.dot_general(h_ref[...], w_ref[...], (((1,), (1,)), ((), ())),
                         preferred_element_type=jnp.float32) + b_ref[...]
    m = jnp.mean(hh, axis=0, keepdims=True)
    v = jnp.mean((hh - m) ** 2, axis=0, keepdims=True)
    hh = jnp.maximum((hh - m) / jnp.sqrt(v + 1e-5) * g_ref[...] + be_ref[...],
                     0.0)
    lg = lax.dot_general(hh, wf_ref[...], (((1,), (1,)), ((), ())),
                         preferred_element_type=jnp.float32) + bf_ref[...]
    mx = jnp.max(lg, axis=1, keepdims=True)
    el = jnp.exp(lg - mx)
    sm = el / jnp.sum(el, axis=1, keepdims=True)
    o_ref[...] = sm[:, 1:]


def _final_stage(h, W, b, g, be, WF, bF):
    return pl.pallas_call(
        _final_body,
        out_shape=jax.ShapeDtypeStruct((N, 1), jnp.float32),
    )(h, W, b.reshape(1, C), g.reshape(1, C), be.reshape(1, C),
      WF, bF.reshape(1, 2))


_NW = 32
_EPW = E // _NW
_CH = 40
_NCHUNK = _EPW // _CH


_CHG = 24
_NPAIR = _EPW // (2 * _CHG)
_TAIL = _EPW - _NPAIR * 2 * _CHG


def _zsum_body(xl_hbm, xr_hbm, src_hbm, dst_hbm, z_hbm,
               sidx, didx, xlb0, xrb0, xlb1, xrb1, sema, semb):
    w = lax.axis_index("s") * 2 + lax.axis_index("c")
    base = w * _EPW
    pltpu.sync_copy(src_hbm.at[pl.ds(base, _EPW)], sidx)
    pltpu.sync_copy(dst_hbm.at[pl.ds(base, _EPW)], didx)

    def fire(o, xlb, xrb, sem):
        pltpu.async_copy(xl_hbm.at[sidx.at[pl.ds(o, _CHG)]], xlb, sem)
        pltpu.async_copy(xr_hbm.at[didx.at[pl.ds(o, _CHG)]], xrb, sem)

    def drain(xlb, sem):
        pltpu.make_async_copy(xl_hbm.at[pl.ds(0, _CHG)], xlb, sem).wait()
        pltpu.make_async_copy(xl_hbm.at[pl.ds(0, _CHG)], xlb, sem).wait()

    def compute(o, xlb, xrb):
        def row(rr, c2):
            for j in range(HC // 16):
                xlb[rr, pl.ds(j * 16, 16)] = (xlb[rr, pl.ds(j * 16, 16)]
                                              + xrb[rr, pl.ds(j * 16, 16)])
            return c2
        lax.fori_loop(0, _CHG, row, 0, unroll=False)
        pltpu.sync_copy(xlb, z_hbm.at[pl.ds(base + o, _CHG)])

    fire(0, xlb0, xrb0, sema)
    fire(_CHG, xlb1, xrb1, semb)
    last_safe = _EPW - 2 * _CHG

    def pair(ci2, c):
        o0 = ci2 * 2 * _CHG
        o1 = o0 + _CHG
        drain(xlb0, sema)
        compute(o0, xlb0, xrb0)
        fire(jnp.minimum(o0 + 2 * _CHG, last_safe), xlb0, xrb0, sema)
        drain(xlb1, semb)
        compute(o1, xlb1, xrb1)
        fire(jnp.minimum(o1 + 2 * _CHG, last_safe), xlb1, xrb1, semb)
        return c

    lax.fori_loop(0, _NPAIR, pair, 0, unroll=False)
    drain(xlb0, sema)
    drain(xlb1, semb)

    to = _NPAIR * 2 * _CHG
    pltpu.async_copy(xl_hbm.at[sidx.at[pl.ds(to, _TAIL)]],
                     xlb0.at[pl.ds(0, _TAIL)], sema).wait()
    pltpu.async_copy(xr_hbm.at[didx.at[pl.ds(to, _TAIL)]],
                     xrb0.at[pl.ds(0, _TAIL)], semb).wait()

    def trow(rr, c2):
        for j in range(HC // 16):
            xlb0[rr, pl.ds(j * 16, 16)] = (xlb0[rr, pl.ds(j * 16, 16)]
                                           + xrb0[rr, pl.ds(j * 16, 16)])
        return c2
    lax.fori_loop(0, _TAIL, trow, 0, unroll=False)
    pltpu.sync_copy(xlb0.at[pl.ds(0, _TAIL)], z_hbm.at[pl.ds(base + to, _TAIL)])


_EPAD = E + 128


def _zsum_sc(xl, xr, ssrc, sdst, offs):
    del offs
    mesh = plsc.VectorSubcoreMesh(core_axis_name="c", subcore_axis_name="s")
    k = functools.partial(
        pl.kernel,
        mesh=mesh,
        compiler_params=pltpu.CompilerParams(needs_layout_passes=False),
        out_type=jax.ShapeDtypeStruct((_EPAD, HC), jnp.float32),
        scratch_types=[
            pltpu.VMEM((_EPW,), jnp.int32),
            pltpu.VMEM((_EPW,), jnp.int32),
            pltpu.VMEM((_CHG, HC), jnp.float32),
            pltpu.VMEM((_CHG, HC), jnp.float32),
            pltpu.VMEM((_CHG, HC), jnp.float32),
            pltpu.VMEM((_CHG, HC), jnp.float32),
            pltpu.SemaphoreType.DMA,
            pltpu.SemaphoreType.DMA,
        ],
    )(_zsum_body)
    return k(xl, xr, ssrc, sdst)


_NR = 160
_RN = 64
_RPW = _NR // _NW
_CHS = 40
_CHA = 96
_NPAD = _NR * _RN
_I16 = lambda: lax.iota(jnp.int32, 16)


def _bcast16(v):
    return jnp.full((16,), v, jnp.int32)


def _scalar(ref, i):
    v = plsc.load_gather(ref, [_bcast16(i)])
    return lax.reduce_max(v, axes=(0,))


def _range_bounds(offsbuf, r):
    lo = _scalar(offsbuf, r)
    hi = _scalar(offsbuf, r + 1)
    return lo, hi


def _mea_body(mepk_hbm, offs_hbm, macc_hbm, offsbuf, mbuf, macc, sem):
    w = lax.axis_index("s") * 2 + lax.axis_index("c")
    pltpu.sync_copy(offs_hbm, offsbuf)
    m3 = _I16() < 3
    zeros = jnp.zeros((16,), jnp.float32)
    for half in range(_RPW):
        r = _RPW * w + half
        nodebase = r * _RN
        lo, hi = _range_bounds(offsbuf, r)

        def zrow(n, c):
            plsc.store_scatter(macc, [_bcast16(n), _I16()], zeros)
            return c
        lax.fori_loop(0, _RN, zrow, 0, unroll=False)

        lo8 = (lo // 8) * 8
        nch = (hi - lo8 + _CHS - 1) // _CHS

        def chunk(ci, c):
            base = lo8 + ci * _CHS
            pltpu.sync_copy(mepk_hbm.at[pl.ds(base, _CHS)], mbuf)
            start = jnp.maximum(lo - base, 0)
            cnt = jnp.minimum(hi - base, _CHS)

            def edge(e, c2):
                av = plsc.load_gather(mbuf, [_bcast16(e), _I16()])
                db = plsc.bitcast(
                    plsc.load_gather(mbuf, [_bcast16(e), _bcast16(8)]),
                    jnp.int32)
                row = db - nodebase
                cur = plsc.load_gather(macc, [row, _I16()])
                plsc.store_scatter(macc, [row, _I16()], cur + av, mask=m3)
                return c2
            lax.fori_loop(start, cnt, edge, 0, unroll=False)
            return c
        lax.fori_loop(0, nch, chunk, 0, unroll=False)
        pltpu.sync_copy(macc, macc_hbm.at[pl.ds(nodebase, _RN)])


def _mea_sc(mepk, offs):
    mesh = plsc.VectorSubcoreMesh(core_axis_name="c", subcore_axis_name="s")
    k = functools.partial(
        pl.kernel,
        mesh=mesh,
        out_type=jax.ShapeDtypeStruct((_NPAD, 16), jnp.float32),
        compiler_params=pltpu.CompilerParams(needs_layout_passes=False),
        scratch_types=[
            pltpu.VMEM((168,), jnp.int32),
            pltpu.VMEM((_CHS, 16), jnp.float32),
            pltpu.VMEM((_RN, 16), jnp.float32),
            pltpu.SemaphoreType.DMA,
        ],
    )(_mea_body)
    return k(mepk, offs)


def _sg_body(aepk_hbm, z_hbm, aself_hbm, offs_hbm,
             agg_hbm, aso_hbm, slo_hbm,
             offsbuf, abuf, zbuf, maxacc, sumacc, aselfbuf, asbuf, slbuf,
             acc, wbuf, sem):
    w = lax.axis_index("s") * 2 + lax.axis_index("c")
    pltpu.sync_copy(offs_hbm, offsbuf)
    m8 = _I16() < 8
    zeros = jnp.zeros((16,), jnp.float32)

    for half in range(_RPW):
        r = _RPW * w + half
        nodebase = r * _RN
        lo, hi = _range_bounds(offsbuf, r)
        lo8 = (lo // 8) * 8
        nch = (hi - lo8 + _CHS - 1) // _CHS
        ncha = (hi - lo8 + _CHA - 1) // _CHA

        pltpu.sync_copy(aself_hbm.at[pl.ds(nodebase, _RN)], maxacc)
        pltpu.sync_copy(aself_hbm.at[pl.ds(nodebase, _RN)], aselfbuf)

        def zrow(n, c):
            plsc.store_scatter(sumacc, [_bcast16(n), _I16()], zeros)
            for j in range(HC // 16):
                plsc.store_scatter(acc, [_bcast16(n), _I16() + j * 16], zeros)
            return c
        lax.fori_loop(0, _RN, zrow, 0, unroll=False)

        def chunk1(ci, c):
            base = lo8 + ci * _CHA
            pltpu.sync_copy(aepk_hbm.at[pl.ds(base, _CHA)], abuf)
            start = jnp.maximum(lo - base, 0)
            cnt = jnp.minimum(hi - base, _CHA)

            def edge(e, c2):
                av = plsc.load_gather(abuf, [_bcast16(e), _I16()])
                db = plsc.bitcast(
                    plsc.load_gather(abuf, [_bcast16(e), _bcast16(8)]),
                    jnp.int32)
                row = db - nodebase
                cur = plsc.load_gather(maxacc, [row, _I16()])
                plsc.store_scatter(maxacc, [row, _I16()],
                                   jnp.maximum(cur, av), mask=m8)
                return c2
            lax.fori_loop(start, cnt, edge, 0, unroll=False)
            return c
        lax.fori_loop(0, ncha, chunk1, 0, unroll=False)

        def chunk2(ci, c):
            base = lo8 + ci * _CHA
            pltpu.sync_copy(aepk_hbm.at[pl.ds(base, _CHA)], abuf)
            start = jnp.maximum(lo - base, 0)
            cnt = jnp.minimum(hi - base, _CHA)

            def edge(e, c2):
                av = plsc.load_gather(abuf, [_bcast16(e), _I16()])
                db = plsc.bitcast(
                    plsc.load_gather(abuf, [_bcast16(e), _bcast16(8)]),
                    jnp.int32)
                row = db - nodebase
                mx = plsc.load_gather(maxacc, [row, _I16()])
                pv = jnp.exp(av - mx)
                cur = plsc.load_gather(sumacc, [row, _I16()])
                plsc.store_scatter(sumacc, [row, _I16()], cur + pv, mask=m8)
                return c2
            lax.fori_loop(start, cnt, edge, 0, unroll=False)
            return c
        lax.fori_loop(0, ncha, chunk2, 0, unroll=False)

        def node(n, c):
            nb = _bcast16(n)
            aself = plsc.load_gather(aselfbuf, [nb, _I16()])
            mx = plsc.load_gather(maxacc, [nb, _I16()])
            sm = plsc.load_gather(sumacc, [nb, _I16()])
            pself = jnp.exp(aself - mx)
            inv = 1.0 / (sm + pself + 1e-16)
            plsc.store_scatter(sumacc, [nb, _I16()], inv, mask=m8)
            plsc.store_scatter(asbuf, [nb, _I16()], pself * inv)
            plsc.store_scatter(slbuf, [nb, _I16()], sm * inv)
            return c
        lax.fori_loop(0, _RN, node, 0, unroll=False)

        def chunk3(ci, c):
            base = lo8 + ci * _CHS
            pltpu.sync_copy(aepk_hbm.at[pl.ds(base, _CHS)], abuf.at[pl.ds(0, _CHS)])
            pltpu.sync_copy(z_hbm.at[pl.ds(base, _CHS)], zbuf)
            start = jnp.maximum(lo - base, 0)
            cnt = jnp.minimum(hi - base, _CHS)

            def edge(e, c2):
                eb = _bcast16(e)
                db = plsc.bitcast(
                    plsc.load_gather(abuf, [eb, _bcast16(8)]), jnp.int32)
                row = db - nodebase
                ws = []
                for h in range(H):
                    hv = _bcast16(h)
                    avh = plsc.load_gather(abuf, [eb, hv])
                    mh = plsc.load_gather(maxacc, [row, hv])
                    ivh = plsc.load_gather(sumacc, [row, hv])
                    ws.append(jnp.exp(avh - mh) * ivh)
                for j in range(HC // 16):
                    col = _I16() + j * 16
                    zv = plsc.load_gather(zbuf, [eb, col])
                    plsc.addupdate_scatter(acc, [row, col], zv * ws[j // 4])
                return c2
            lax.fori_loop(start, cnt, edge, 0, unroll=False)
            return c
        lax.fori_loop(0, nch, chunk3, 0, unroll=False)

        pltpu.sync_copy(acc, agg_hbm.at[pl.ds(nodebase, _RN)])
        pltpu.sync_copy(asbuf, aso_hbm.at[pl.ds(nodebase, _RN)])
        pltpu.sync_copy(slbuf, slo_hbm.at[pl.ds(nodebase, _RN)])


def _sg_sc(aepk, zsum, aselfpk, offs):
    mesh = plsc.VectorSubcoreMesh(core_axis_name="c", subcore_axis_name="s")
    k = functools.partial(
        pl.kernel,
        mesh=mesh,
        compiler_params=pltpu.CompilerParams(needs_layout_passes=False),
        out_type=(jax.ShapeDtypeStruct((_NPAD, HC), jnp.float32),
                  jax.ShapeDtypeStruct((_NPAD, 16), jnp.float32),
                  jax.ShapeDtypeStruct((_NPAD, 16), jnp.float32)),
        scratch_types=[
            pltpu.VMEM((168,), jnp.int32),
            pltpu.VMEM((_CHA, 16), jnp.float32),
            pltpu.VMEM((_CHS, HC), jnp.float32),
            pltpu.VMEM((_RN, 16), jnp.float32),
            pltpu.VMEM((_RN, 16), jnp.float32),
            pltpu.VMEM((_RN, 16), jnp.float32),
            pltpu.VMEM((_RN, 16), jnp.float32),
            pltpu.VMEM((_RN, 16), jnp.float32),
            pltpu.VMEM((_RN, HC), jnp.float32),
            pltpu.VMEM((16,), jnp.float32),
            pltpu.SemaphoreType.DMA,
        ],
    )(_sg_body)
    return k(aepk, zsum, aselfpk, offs)


_EB = 2000


def _escore_body(z_ref, ea_ref, we_ref, attf_ref, sel_ref, m_ref, d_ref,
                 o_ref):
    ee = lax.dot_general(ea_ref[...], we_ref[...], (((1,), (1,)), ((), ())),
                         preferred_element_type=jnp.float32)
    z = z_ref[...] + ee
    z = jnp.where(z >= 0.0, z, 0.2 * z)
    a = lax.dot_general(z * attf_ref[...], sel_ref[...],
                        (((1,), (0,)), ((), ())),
                        preferred_element_type=jnp.float32)
    a = jnp.where(m_ref[...] > 0.0, a, -jnp.inf)
    zpad = jnp.zeros((a.shape[0], 7), jnp.float32)
    o_ref[...] = jnp.concatenate([a, d_ref[...], zpad], axis=1)


def _edge_scores(zsum, sea, We, att, maskf, sdstf):
    attf = att.reshape(1, HC)
    sel = jnp.zeros((HC, H), jnp.float32)
    sel = sel.at[jnp.arange(HC), jnp.arange(HC) // C].set(1.0)
    nb = E // _EB
    return pl.pallas_call(
        _escore_body,
        grid=(nb,),
        in_specs=[
            pl.BlockSpec((_EB, HC), lambda i: (i, 0)),
            pl.BlockSpec((_EB, 2), lambda i: (i, 0)),
            pl.BlockSpec((HC, 2), lambda i: (0, 0)),
            pl.BlockSpec((1, HC), lambda i: (0, 0)),
            pl.BlockSpec((HC, H), lambda i: (0, 0)),
            pl.BlockSpec((_EB, 1), lambda i: (i, 0)),
            pl.BlockSpec((_EB, 1), lambda i: (i, 0)),
        ],
        out_specs=pl.BlockSpec((_EB, 16), lambda i: (i, 0)),
        out_shape=jax.ShapeDtypeStruct((E, 16), jnp.float32),
    )(zsum, sea, We, attf, sel, maskf.reshape(E, 1), sdstf.reshape(E, 1))


def _combine_body(agg_ref, xl_ref, xr_ref, as_ref, sl_ref, b_ref, exp_ref,
                  o_ref):
    aexp = lax.dot_general(as_ref[...], exp_ref[...], (((1,), (0,)), ((), ())),
                           preferred_element_type=jnp.float32)
    sexp = lax.dot_general(sl_ref[...], exp_ref[...], (((1,), (0,)), ((), ())),
                           preferred_element_type=jnp.float32)
    out = (agg_ref[...] - sexp * xr_ref[...] + aexp * xl_ref[...]
           + b_ref[...])
    o_ref[...] = jnp.maximum(out, 0.0)


def _combine(agg, xl, xr, aself_o, salpha_o, bias):
    exp16 = jnp.zeros((16, HC), jnp.float32)
    exp16 = exp16.at[jnp.arange(HC) // C, jnp.arange(HC)].set(1.0)
    nb = N // _RB
    return pl.pallas_call(
        _combine_body,
        grid=(nb,),
        in_specs=[
            pl.BlockSpec((_RB, HC), lambda i: (i, 0)),
            pl.BlockSpec((_RB, HC), lambda i: (i, 0)),
            pl.BlockSpec((_RB, HC), lambda i: (i, 0)),
            pl.BlockSpec((_RB, 16), lambda i: (i, 0)),
            pl.BlockSpec((_RB, 16), lambda i: (i, 0)),
            pl.BlockSpec((1, HC), lambda i: (0, 0)),
            pl.BlockSpec((16, HC), lambda i: (0, 0)),
        ],
        out_specs=pl.BlockSpec((_RB, HC), lambda i: (i, 0)),
        out_shape=jax.ShapeDtypeStruct((N, HC), jnp.float32),
    )(agg, xl, xr, aself_o, salpha_o, bias.reshape(1, HC), exp16)


def _gat_layer(h, ssrc, sdst, sdstf, sea, offs, p, smaskf):
    xl, xr = _gat_proj(h, p)

    mea = sea * smaskf[:, None]
    mepk = jnp.concatenate(
        [mea, smaskf[:, None], jnp.zeros((E, 5), jnp.float32),
         sdstf[:, None], jnp.zeros((E, 7), jnp.float32)], axis=1)
    mepk = jnp.concatenate([mepk, jnp.zeros((128, 16), jnp.float32)], axis=0)
    macc = _mea_sc(mepk, offs)
    mean_ea = macc[:N, 0:2] / jnp.maximum(macc[:N, 2:3], 1.0)

    a_self = _self_scores(xl, xr, mean_ea, p["We"], p["att"])
    aselfpk = jnp.pad(a_self, ((0, _NPAD - N), (0, 8)))

    zsum = _zsum_sc(xl, xr, ssrc, sdst, offs)
    a16 = _edge_scores(zsum, sea, p["We"], p["att"], smaskf, sdstf)
    aepk = jnp.concatenate([a16, jnp.zeros((128, 16), jnp.float32)], axis=0)

    agg, aself_o, salpha_o = _sg_sc(aepk, zsum, aselfpk, offs)

    return _combine(agg, xl, xr, aself_o, salpha_o, p["bias"])


def kernel(x, edge_index, edge_attr, params):
    src = edge_index[0]
    dst = edge_index[1]
    a0 = edge_attr[:, 0]
    fea = edge_attr[:, 1:]
    p = params

    perm = jnp.argsort(dst)
    ssrc = src[perm]
    sdst = dst[perm]
    sea = fea[perm]
    sa0 = a0[perm]
    sdstf = lax.bitcast_convert_type(sdst, jnp.float32)
    bounds = jnp.minimum(jnp.arange(_NR + 1, dtype=jnp.int32) * _RN, N)
    offs = jnp.searchsorted(sdst, bounds, side="left").astype(jnp.int32)
    offs = jnp.concatenate([offs, jnp.zeros((168 - _NR - 1,), jnp.int32)])
    m1 = (sa0 >= 0).astype(jnp.float32)
    m2 = (sa0 <= 0).astype(jnp.float32)
    mall = jnp.ones((E,), jnp.float32)

    h = _lin_bn_relu(x, p["W0"], p["b0"], p["g0"], p["be0"])
    h = _gat_layer(h, ssrc, sdst, sdstf, sea, offs, p["gat1"], m1)
    h = _lin_bn_relu(h, p["W1"], p["b1"], p["g1"], p["be1"])
    h = _gat_layer(h, ssrc, sdst, sdstf, sea, offs, p["gat2"], m2)
    h = _lin_bn_relu(h, p["W2"], p["b2"], p["g2"], p["be2"])
    h = _gat_layer(h, ssrc, sdst, sdstf, sea, offs, p["gat3"], mall)
    return _final_stage(h, p["W3"], p["b3"], p["g3"], p["be3"],
                        p["WF"], p["bF"])


# speedup vs baseline: 8.5474x; 1.0003x over previous
import functools

import jax
import jax.numpy as jnp
from jax import lax
from jax.experimental import pallas as pl
from jax.experimental.pallas import tpu as pltpu
from jax.experimental.pallas import tpu_sc as plsc

N = 10000
E = 160000
FEAT = 256
C = 64
H = 8
HC = H * C


def _linbn_body(h_ref, w_ref, b_ref, g_ref, be_ref, o_ref):
    hh = lax.dot_general(h_ref[...], w_ref[...],
                         (((1,), (1,)), ((), ())),
                         preferred_element_type=jnp.float32)
    hh = hh + b_ref[...]
    m = jnp.mean(hh, axis=0, keepdims=True)
    v = jnp.mean((hh - m) ** 2, axis=0, keepdims=True)
    o_ref[...] = jnp.maximum((hh - m) / jnp.sqrt(v + 1e-5) * g_ref[...]
                             + be_ref[...], 0.0)


def _lin_bn_relu(h, W, b, g, be):
    n, _ = h.shape
    co = W.shape[0]
    return pl.pallas_call(
        _linbn_body,
        out_shape=jax.ShapeDtypeStruct((n, co), jnp.float32),
    )(h, W, b.reshape(1, co), g.reshape(1, co), be.reshape(1, co))


def _proj_body(h_ref, wl_ref, bl_ref, wr_ref, br_ref, xl_ref, xr_ref):
    h = h_ref[...]
    xl_ref[...] = lax.dot_general(h, wl_ref[...], (((1,), (1,)), ((), ())),
                                  preferred_element_type=jnp.float32) + bl_ref[...]
    xr_ref[...] = lax.dot_general(h, wr_ref[...], (((1,), (1,)), ((), ())),
                                  preferred_element_type=jnp.float32) + br_ref[...]


_RB = 2000


def _gat_proj(h, p):
    n = h.shape[0]
    nb = n // _RB
    return pl.pallas_call(
        _proj_body,
        grid=(nb,),
        in_specs=[
            pl.BlockSpec((_RB, C), lambda i: (i, 0)),
            pl.BlockSpec((HC, C), lambda i: (0, 0)),
            pl.BlockSpec((1, HC), lambda i: (0, 0)),
            pl.BlockSpec((HC, C), lambda i: (0, 0)),
            pl.BlockSpec((1, HC), lambda i: (0, 0)),
        ],
        out_specs=(pl.BlockSpec((_RB, HC), lambda i: (i, 0)),
                   pl.BlockSpec((_RB, HC), lambda i: (i, 0))),
        out_shape=(jax.ShapeDtypeStruct((n, HC), jnp.float32),
                   jax.ShapeDtypeStruct((n, HC), jnp.float32)),
    )(h, p["Wl"], p["bl"].reshape(1, HC), p["Wr"], p["br"].reshape(1, HC))


def _self_score_body(xl_ref, xr_ref, mea_ref, we_ref, attf_ref, sel_ref,
                     a_ref):
    ee = lax.dot_general(mea_ref[...], we_ref[...], (((1,), (1,)), ((), ())),
                         preferred_element_type=jnp.float32)
    z = xl_ref[...] + xr_ref[...] + ee
    z = jnp.where(z >= 0.0, z, 0.2 * z)
    za = z * attf_ref[...]
    a_ref[...] = lax.dot_general(za, sel_ref[...], (((1,), (0,)), ((), ())),
                                 preferred_element_type=jnp.float32)


def _self_scores(xl, xr, mean_ea, We, att):
    attf = att.reshape(1, HC)
    sel = jnp.zeros((HC, H), jnp.float32)
    sel = sel.at[jnp.arange(HC), jnp.arange(HC) // C].set(1.0)
    nb = N // _RB
    return pl.pallas_call(
        _self_score_body,
        grid=(nb,),
        in_specs=[
            pl.BlockSpec((_RB, HC), lambda i: (i, 0)),
            pl.BlockSpec((_RB, HC), lambda i: (i, 0)),
            pl.BlockSpec((_RB, 2), lambda i: (i, 0)),
            pl.BlockSpec((HC, 2), lambda i: (0, 0)),
            pl.BlockSpec((1, HC), lambda i: (0, 0)),
            pl.BlockSpec((HC, H), lambda i: (0, 0)),
        ],
        out_specs=pl.BlockSpec((_RB, H), lambda i: (i, 0)),
        out_shape=jax.ShapeDtypeStruct((N, H), jnp.float32),
    )(xl, xr, mean_ea, We, attf, sel)


def _final_body(h_ref, w_ref, b_ref, g_ref, be_ref, wf_ref, bf_ref, o_ref):
    hh = lax.dot_general(h_ref[...], w_ref[...], (((1,), (1,)), ((), ())),
                         preferred_element_type=jnp.float32) + b_ref[...]
    m = jnp.mean(hh, axis=0, keepdims=True)
    v = jnp.mean((hh - m) ** 2, axis=0, keepdims=True)
    hh = jnp.maximum((hh - m) / jnp.sqrt(v + 1e-5) * g_ref[...] + be_ref[...],
                     0.0)
    lg = lax.dot_general(hh, wf_ref[...], (((1,), (1,)), ((), ())),
                         preferred_element_type=jnp.float32) + bf_ref[...]
    mx = jnp.max(lg, axis=1, keepdims=True)
    el = jnp.exp(lg - mx)
    sm = el / jnp.sum(el, axis=1, keepdims=True)
    o_ref[...] = sm[:, 1:]


def _final_stage(h, W, b, g, be, WF, bF):
    return pl.pallas_call(
        _final_body,
        out_shape=jax.ShapeDtypeStruct((N, 1), jnp.float32),
    )(h, W, b.reshape(1, C), g.reshape(1, C), be.reshape(1, C),
      WF, bF.reshape(1, 2))


_NW = 32
_EPW = E // _NW
_CH = 40
_NCHUNK = _EPW // _CH


_CHG = 24
_NPAIR = _EPW // (2 * _CHG)
_TAIL = _EPW - _NPAIR * 2 * _CHG


def _zsum_body(xl_hbm, xr_hbm, src_hbm, dst_hbm, z_hbm,
               sidx, didx, xlb0, xrb0, xlb1, xrb1, sema, semb):
    w = lax.axis_index("s") * 2 + lax.axis_index("c")
    base = w * _EPW
    pltpu.sync_copy(src_hbm.at[pl.ds(base, _EPW)], sidx)
    pltpu.sync_copy(dst_hbm.at[pl.ds(base, _EPW)], didx)

    def fire(o, xlb, xrb, sem):
        pltpu.async_copy(xl_hbm.at[sidx.at[pl.ds(o, _CHG)]], xlb, sem)
        pltpu.async_copy(xr_hbm.at[didx.at[pl.ds(o, _CHG)]], xrb, sem)

    def drain(xlb, sem):
        pltpu.make_async_copy(xl_hbm.at[pl.ds(0, _CHG)], xlb, sem).wait()
        pltpu.make_async_copy(xl_hbm.at[pl.ds(0, _CHG)], xlb, sem).wait()

    def compute(o, xlb, xrb):
        def row(rr, c2):
            for j in range(HC // 16):
                xlb[rr, pl.ds(j * 16, 16)] = (xlb[rr, pl.ds(j * 16, 16)]
                                              + xrb[rr, pl.ds(j * 16, 16)])
            return c2
        lax.fori_loop(0, _CHG, row, 0, unroll=False)
        pltpu.sync_copy(xlb, z_hbm.at[pl.ds(base + o, _CHG)])

    fire(0, xlb0, xrb0, sema)
    fire(_CHG, xlb1, xrb1, semb)
    last_safe = _EPW - _CHG

    def pair(ci2, c):
        o0 = ci2 * 2 * _CHG
        o1 = o0 + _CHG
        drain(xlb0, sema)
        compute(o0, xlb0, xrb0)
        fire(jnp.minimum(o0 + 2 * _CHG, last_safe), xlb0, xrb0, sema)
        drain(xlb1, semb)
        compute(o1, xlb1, xrb1)
        fire(jnp.minimum(o1 + 2 * _CHG, last_safe), xlb1, xrb1, semb)
        return c

    lax.fori_loop(0, _NPAIR, pair, 0, unroll=False)
    drain(xlb0, sema)
    drain(xlb1, semb)

    to = _NPAIR * 2 * _CHG
    pltpu.async_copy(xl_hbm.at[sidx.at[pl.ds(to, _TAIL)]],
                     xlb0.at[pl.ds(0, _TAIL)], sema).wait()
    pltpu.async_copy(xr_hbm.at[didx.at[pl.ds(to, _TAIL)]],
                     xrb0.at[pl.ds(0, _TAIL)], semb).wait()

    def trow(rr, c2):
        for j in range(HC // 16):
            xlb0[rr, pl.ds(j * 16, 16)] = (xlb0[rr, pl.ds(j * 16, 16)]
                                           + xrb0[rr, pl.ds(j * 16, 16)])
        return c2
    lax.fori_loop(0, _TAIL, trow, 0, unroll=False)
    pltpu.sync_copy(xlb0.at[pl.ds(0, _TAIL)], z_hbm.at[pl.ds(base + to, _TAIL)])


_EPAD = E + 128


def _zsum_sc(xl, xr, ssrc, sdst, offs):
    del offs
    mesh = plsc.VectorSubcoreMesh(core_axis_name="c", subcore_axis_name="s")
    k = functools.partial(
        pl.kernel,
        mesh=mesh,
        compiler_params=pltpu.CompilerParams(needs_layout_passes=False),
        out_type=jax.ShapeDtypeStruct((_EPAD, HC), jnp.float32),
        scratch_types=[
            pltpu.VMEM((_EPW,), jnp.int32),
            pltpu.VMEM((_EPW,), jnp.int32),
            pltpu.VMEM((_CHG, HC), jnp.float32),
            pltpu.VMEM((_CHG, HC), jnp.float32),
            pltpu.VMEM((_CHG, HC), jnp.float32),
            pltpu.VMEM((_CHG, HC), jnp.float32),
            pltpu.SemaphoreType.DMA,
            pltpu.SemaphoreType.DMA,
        ],
    )(_zsum_body)
    return k(xl, xr, ssrc, sdst)


_NR = 160
_RN = 64
_RPW = _NR // _NW
_CHS = 40
_CHA = 96
_NPAD = _NR * _RN
_I16 = lambda: lax.iota(jnp.int32, 16)


def _bcast16(v):
    return jnp.full((16,), v, jnp.int32)


def _scalar(ref, i):
    v = plsc.load_gather(ref, [_bcast16(i)])
    return lax.reduce_max(v, axes=(0,))


def _range_bounds(offsbuf, r):
    lo = _scalar(offsbuf, r)
    hi = _scalar(offsbuf, r + 1)
    return lo, hi


def _mea_body(mepk_hbm, offs_hbm, macc_hbm, offsbuf, mbuf, macc, sem):
    w = lax.axis_index("s") * 2 + lax.axis_index("c")
    pltpu.sync_copy(offs_hbm, offsbuf)
    m3 = _I16() < 3
    zeros = jnp.zeros((16,), jnp.float32)
    for half in range(_RPW):
        r = _RPW * w + half
        nodebase = r * _RN
        lo, hi = _range_bounds(offsbuf, r)

        def zrow(n, c):
            plsc.store_scatter(macc, [_bcast16(n), _I16()], zeros)
            return c
        lax.fori_loop(0, _RN, zrow, 0, unroll=False)

        lo8 = (lo // 8) * 8
        nch = (hi - lo8 + _CHS - 1) // _CHS

        def chunk(ci, c):
            base = lo8 + ci * _CHS
            pltpu.sync_copy(mepk_hbm.at[pl.ds(base, _CHS)], mbuf)
            start = jnp.maximum(lo - base, 0)
            cnt = jnp.minimum(hi - base, _CHS)

            def edge(e, c2):
                av = plsc.load_gather(mbuf, [_bcast16(e), _I16()])
                db = plsc.bitcast(
                    plsc.load_gather(mbuf, [_bcast16(e), _bcast16(8)]),
                    jnp.int32)
                row = db - nodebase
                cur = plsc.load_gather(macc, [row, _I16()])
                plsc.store_scatter(macc, [row, _I16()], cur + av, mask=m3)
                return c2
            lax.fori_loop(start, cnt, edge, 0, unroll=False)
            return c
        lax.fori_loop(0, nch, chunk, 0, unroll=False)
        pltpu.sync_copy(macc, macc_hbm.at[pl.ds(nodebase, _RN)])


def _mea_sc(mepk, offs):
    mesh = plsc.VectorSubcoreMesh(core_axis_name="c", subcore_axis_name="s")
    k = functools.partial(
        pl.kernel,
        mesh=mesh,
        out_type=jax.ShapeDtypeStruct((_NPAD, 16), jnp.float32),
        compiler_params=pltpu.CompilerParams(needs_layout_passes=False),
        scratch_types=[
            pltpu.VMEM((168,), jnp.int32),
            pltpu.VMEM((_CHS, 16), jnp.float32),
            pltpu.VMEM((_RN, 16), jnp.float32),
            pltpu.SemaphoreType.DMA,
        ],
    )(_mea_body)
    return k(mepk, offs)


def _sg_body(aepk_hbm, z_hbm, aself_hbm, offs_hbm,
             agg_hbm, aso_hbm, slo_hbm,
             offsbuf, abuf, zbuf, maxacc, sumacc, aselfbuf, asbuf, slbuf,
             acc, wbuf, sem):
    w = lax.axis_index("s") * 2 + lax.axis_index("c")
    pltpu.sync_copy(offs_hbm, offsbuf)
    m8 = _I16() < 8
    zeros = jnp.zeros((16,), jnp.float32)

    for half in range(_RPW):
        r = _RPW * w + half
        nodebase = r * _RN
        lo, hi = _range_bounds(offsbuf, r)
        lo8 = (lo // 8) * 8
        nch = (hi - lo8 + _CHS - 1) // _CHS
        ncha = (hi - lo8 + _CHA - 1) // _CHA

        pltpu.sync_copy(aself_hbm.at[pl.ds(nodebase, _RN)], maxacc)
        pltpu.sync_copy(aself_hbm.at[pl.ds(nodebase, _RN)], aselfbuf)

        def zrow(n, c):
            plsc.store_scatter(sumacc, [_bcast16(n), _I16()], zeros)
            for j in range(HC // 16):
                plsc.store_scatter(acc, [_bcast16(n), _I16() + j * 16], zeros)
            return c
        lax.fori_loop(0, _RN, zrow, 0, unroll=False)

        def chunk1(ci, c):
            base = lo8 + ci * _CHA
            pltpu.sync_copy(aepk_hbm.at[pl.ds(base, _CHA)], abuf)
            start = jnp.maximum(lo - base, 0)
            cnt = jnp.minimum(hi - base, _CHA)

            def edge(e, c2):
                av = plsc.load_gather(abuf, [_bcast16(e), _I16()])
                db = plsc.bitcast(
                    plsc.load_gather(abuf, [_bcast16(e), _bcast16(8)]),
                    jnp.int32)
                row = db - nodebase
                cur = plsc.load_gather(maxacc, [row, _I16()])
                plsc.store_scatter(maxacc, [row, _I16()],
                                   jnp.maximum(cur, av), mask=m8)
                return c2
            lax.fori_loop(start, cnt, edge, 0, unroll=False)
            return c
        lax.fori_loop(0, ncha, chunk1, 0, unroll=False)

        def chunk2(ci, c):
            base = lo8 + ci * _CHA
            pltpu.sync_copy(aepk_hbm.at[pl.ds(base, _CHA)], abuf)
            start = jnp.maximum(lo - base, 0)
            cnt = jnp.minimum(hi - base, _CHA)

            def edge(e, c2):
                av = plsc.load_gather(abuf, [_bcast16(e), _I16()])
                db = plsc.bitcast(
                    plsc.load_gather(abuf, [_bcast16(e), _bcast16(8)]),
                    jnp.int32)
                row = db - nodebase
                mx = plsc.load_gather(maxacc, [row, _I16()])
                pv = jnp.exp(av - mx)
                cur = plsc.load_gather(sumacc, [row, _I16()])
                plsc.store_scatter(sumacc, [row, _I16()], cur + pv, mask=m8)
                return c2
            lax.fori_loop(start, cnt, edge, 0, unroll=False)
            return c
        lax.fori_loop(0, ncha, chunk2, 0, unroll=False)

        def node(n, c):
            nb = _bcast16(n)
            aself = plsc.load_gather(aselfbuf, [nb, _I16()])
            mx = plsc.load_gather(maxacc, [nb, _I16()])
            sm = plsc.load_gather(sumacc, [nb, _I16()])
            pself = jnp.exp(aself - mx)
            inv = 1.0 / (sm + pself + 1e-16)
            plsc.store_scatter(sumacc, [nb, _I16()], inv, mask=m8)
            plsc.store_scatter(asbuf, [nb, _I16()], pself * inv)
            plsc.store_scatter(slbuf, [nb, _I16()], sm * inv)
            return c
        lax.fori_loop(0, _RN, node, 0, unroll=False)

        def chunk3(ci, c):
            base = lo8 + ci * _CHS
            pltpu.sync_copy(aepk_hbm.at[pl.ds(base, _CHS)], abuf.at[pl.ds(0, _CHS)])
            pltpu.sync_copy(z_hbm.at[pl.ds(base, _CHS)], zbuf)
            start = jnp.maximum(lo - base, 0)
            cnt = jnp.minimum(hi - base, _CHS)

            def edge(e, c2):
                eb = _bcast16(e)
                db = plsc.bitcast(
                    plsc.load_gather(abuf, [eb, _bcast16(8)]), jnp.int32)
                row = db - nodebase
                ws = []
                for h in range(H):
                    hv = _bcast16(h)
                    avh = plsc.load_gather(abuf, [eb, hv])
                    mh = plsc.load_gather(maxacc, [row, hv])
                    ivh = plsc.load_gather(sumacc, [row, hv])
                    ws.append(jnp.exp(avh - mh) * ivh)
                for j in range(HC // 16):
                    col = _I16() + j * 16
                    zv = plsc.load_gather(zbuf, [eb, col])
                    plsc.addupdate_scatter(acc, [row, col], zv * ws[j // 4])
                return c2
            lax.fori_loop(start, cnt, edge, 0, unroll=False)
            return c
        lax.fori_loop(0, nch, chunk3, 0, unroll=False)

        pltpu.sync_copy(acc, agg_hbm.at[pl.ds(nodebase, _RN)])
        pltpu.sync_copy(asbuf, aso_hbm.at[pl.ds(nodebase, _RN)])
        pltpu.sync_copy(slbuf, slo_hbm.at[pl.ds(nodebase, _RN)])


def _sg_sc(aepk, zsum, aselfpk, offs):
    mesh = plsc.VectorSubcoreMesh(core_axis_name="c", subcore_axis_name="s")
    k = functools.partial(
        pl.kernel,
        mesh=mesh,
        compiler_params=pltpu.CompilerParams(needs_layout_passes=False),
        out_type=(jax.ShapeDtypeStruct((_NPAD, HC), jnp.float32),
                  jax.ShapeDtypeStruct((_NPAD, 16), jnp.float32),
                  jax.ShapeDtypeStruct((_NPAD, 16), jnp.float32)),
        scratch_types=[
            pltpu.VMEM((168,), jnp.int32),
            pltpu.VMEM((_CHA, 16), jnp.float32),
            pltpu.VMEM((_CHS, HC), jnp.float32),
            pltpu.VMEM((_RN, 16), jnp.float32),
            pltpu.VMEM((_RN, 16), jnp.float32),
            pltpu.VMEM((_RN, 16), jnp.float32),
            pltpu.VMEM((_RN, 16), jnp.float32),
            pltpu.VMEM((_RN, 16), jnp.float32),
            pltpu.VMEM((_RN, HC), jnp.float32),
            pltpu.VMEM((16,), jnp.float32),
            pltpu.SemaphoreType.DMA,
        ],
    )(_sg_body)
    return k(aepk, zsum, aselfpk, offs)


_EB = 2000


def _escore_body(z_ref, ea_ref, we_ref, attf_ref, sel_ref, m_ref, d_ref,
                 o_ref):
    ee = lax.dot_general(ea_ref[...], we_ref[...], (((1,), (1,)), ((), ())),
                         preferred_element_type=jnp.float32)
    z = z_ref[...] + ee
    z = jnp.where(z >= 0.0, z, 0.2 * z)
    a = lax.dot_general(z * attf_ref[...], sel_ref[...],
                        (((1,), (0,)), ((), ())),
                        preferred_element_type=jnp.float32)
    a = jnp.where(m_ref[...] > 0.0, a, -jnp.inf)
    zpad = jnp.zeros((a.shape[0], 7), jnp.float32)
    o_ref[...] = jnp.concatenate([a, d_ref[...], zpad], axis=1)


def _edge_scores(zsum, sea, We, att, maskf, sdstf):
    attf = att.reshape(1, HC)
    sel = jnp.zeros((HC, H), jnp.float32)
    sel = sel.at[jnp.arange(HC), jnp.arange(HC) // C].set(1.0)
    nb = E // _EB
    return pl.pallas_call(
        _escore_body,
        grid=(nb,),
        in_specs=[
            pl.BlockSpec((_EB, HC), lambda i: (i, 0)),
            pl.BlockSpec((_EB, 2), lambda i: (i, 0)),
            pl.BlockSpec((HC, 2), lambda i: (0, 0)),
            pl.BlockSpec((1, HC), lambda i: (0, 0)),
            pl.BlockSpec((HC, H), lambda i: (0, 0)),
            pl.BlockSpec((_EB, 1), lambda i: (i, 0)),
            pl.BlockSpec((_EB, 1), lambda i: (i, 0)),
        ],
        out_specs=pl.BlockSpec((_EB, 16), lambda i: (i, 0)),
        out_shape=jax.ShapeDtypeStruct((E, 16), jnp.float32),
    )(zsum, sea, We, attf, sel, maskf.reshape(E, 1), sdstf.reshape(E, 1))


def _combine_body(agg_ref, xl_ref, xr_ref, as_ref, sl_ref, b_ref, exp_ref,
                  o_ref):
    aexp = lax.dot_general(as_ref[...], exp_ref[...], (((1,), (0,)), ((), ())),
                           preferred_element_type=jnp.float32)
    sexp = lax.dot_general(sl_ref[...], exp_ref[...], (((1,), (0,)), ((), ())),
                           preferred_element_type=jnp.float32)
    out = (agg_ref[...] - sexp * xr_ref[...] + aexp * xl_ref[...]
           + b_ref[...])
    o_ref[...] = jnp.maximum(out, 0.0)


def _combine(agg, xl, xr, aself_o, salpha_o, bias):
    exp16 = jnp.zeros((16, HC), jnp.float32)
    exp16 = exp16.at[jnp.arange(HC) // C, jnp.arange(HC)].set(1.0)
    nb = N // _RB
    return pl.pallas_call(
        _combine_body,
        grid=(nb,),
        in_specs=[
            pl.BlockSpec((_RB, HC), lambda i: (i, 0)),
            pl.BlockSpec((_RB, HC), lambda i: (i, 0)),
            pl.BlockSpec((_RB, HC), lambda i: (i, 0)),
            pl.BlockSpec((_RB, 16), lambda i: (i, 0)),
            pl.BlockSpec((_RB, 16), lambda i: (i, 0)),
            pl.BlockSpec((1, HC), lambda i: (0, 0)),
            pl.BlockSpec((16, HC), lambda i: (0, 0)),
        ],
        out_specs=pl.BlockSpec((_RB, HC), lambda i: (i, 0)),
        out_shape=jax.ShapeDtypeStruct((N, HC), jnp.float32),
    )(agg, xl, xr, aself_o, salpha_o, bias.reshape(1, HC), exp16)


def _gat_layer(h, ssrc, sdst, sdstf, sea, offs, p, smaskf):
    xl, xr = _gat_proj(h, p)

    mea = sea * smaskf[:, None]
    mepk = jnp.concatenate(
        [mea, smaskf[:, None], jnp.zeros((E, 5), jnp.float32),
         sdstf[:, None], jnp.zeros((E, 7), jnp.float32)], axis=1)
    mepk = jnp.concatenate([mepk, jnp.zeros((128, 16), jnp.float32)], axis=0)
    macc = _mea_sc(mepk, offs)
    mean_ea = macc[:N, 0:2] / jnp.maximum(macc[:N, 2:3], 1.0)

    a_self = _self_scores(xl, xr, mean_ea, p["We"], p["att"])
    aselfpk = jnp.pad(a_self, ((0, _NPAD - N), (0, 8)))

    zsum = _zsum_sc(xl, xr, ssrc, sdst, offs)
    a16 = _edge_scores(zsum, sea, p["We"], p["att"], smaskf, sdstf)
    aepk = jnp.concatenate([a16, jnp.zeros((128, 16), jnp.float32)], axis=0)

    agg, aself_o, salpha_o = _sg_sc(aepk, zsum, aselfpk, offs)

    return _combine(agg, xl, xr, aself_o, salpha_o, p["bias"])


def kernel(x, edge_index, edge_attr, params):
    src = edge_index[0]
    dst = edge_index[1]
    a0 = edge_attr[:, 0]
    fea = edge_attr[:, 1:]
    p = params

    perm = jnp.argsort(dst)
    ssrc = src[perm]
    sdst = dst[perm]
    sea = fea[perm]
    sa0 = a0[perm]
    sdstf = lax.bitcast_convert_type(sdst, jnp.float32)
    bounds = jnp.minimum(jnp.arange(_NR + 1, dtype=jnp.int32) * _RN, N)
    offs = jnp.searchsorted(sdst, bounds, side="left").astype(jnp.int32)
    offs = jnp.concatenate([offs, jnp.zeros((168 - _NR - 1,), jnp.int32)])
    m1 = (sa0 >= 0).astype(jnp.float32)
    m2 = (sa0 <= 0).astype(jnp.float32)
    mall = jnp.ones((E,), jnp.float32)

    h = _lin_bn_relu(x, p["W0"], p["b0"], p["g0"], p["be0"])
    h = _gat_layer(h, ssrc, sdst, sdstf, sea, offs, p["gat1"], m1)
    h = _lin_bn_relu(h, p["W1"], p["b1"], p["g1"], p["be1"])
    h = _gat_layer(h, ssrc, sdst, sdstf, sea, offs, p["gat2"], m2)
    h = _lin_bn_relu(h, p["W2"], p["b2"], p["g2"], p["be2"])
    h = _gat_layer(h, ssrc, sdst, sdstf, sea, offs, p["gat3"], mall)
    return _final_stage(h, p["W3"], p["b3"], p["g3"], p["be3"],
                        p["WF"], p["bF"])


# speedup vs baseline: 9.2735x; 1.0850x over previous
import functools

import jax
import jax.numpy as jnp
from jax import lax
from jax.experimental import pallas as pl
from jax.experimental.pallas import tpu as pltpu
from jax.experimental.pallas import tpu_sc as plsc

N = 10000
E = 160000
FEAT = 256
C = 64
H = 8
HC = H * C


def _linbn_body(h_ref, w_ref, b_ref, g_ref, be_ref, o_ref):
    hh = lax.dot_general(h_ref[...], w_ref[...],
                         (((1,), (1,)), ((), ())),
                         preferred_element_type=jnp.float32)
    hh = hh + b_ref[...]
    m = jnp.mean(hh, axis=0, keepdims=True)
    v = jnp.mean((hh - m) ** 2, axis=0, keepdims=True)
    o_ref[...] = jnp.maximum((hh - m) / jnp.sqrt(v + 1e-5) * g_ref[...]
                             + be_ref[...], 0.0)


def _lin_bn_relu(h, W, b, g, be):
    n, _ = h.shape
    co = W.shape[0]
    return pl.pallas_call(
        _linbn_body,
        out_shape=jax.ShapeDtypeStruct((n, co), jnp.float32),
    )(h, W, b.reshape(1, co), g.reshape(1, co), be.reshape(1, co))


def _proj_body(h_ref, wl_ref, bl_ref, wr_ref, br_ref, xl_ref, xr_ref):
    h = h_ref[...]
    xl_ref[...] = lax.dot_general(h, wl_ref[...], (((1,), (1,)), ((), ())),
                                  preferred_element_type=jnp.float32) + bl_ref[...]
    xr_ref[...] = lax.dot_general(h, wr_ref[...], (((1,), (1,)), ((), ())),
                                  preferred_element_type=jnp.float32) + br_ref[...]


_RB = 2000


def _gat_proj(h, p):
    n = h.shape[0]
    nb = n // _RB
    return pl.pallas_call(
        _proj_body,
        grid=(nb,),
        in_specs=[
            pl.BlockSpec((_RB, C), lambda i: (i, 0)),
            pl.BlockSpec((HC, C), lambda i: (0, 0)),
            pl.BlockSpec((1, HC), lambda i: (0, 0)),
            pl.BlockSpec((HC, C), lambda i: (0, 0)),
            pl.BlockSpec((1, HC), lambda i: (0, 0)),
        ],
        out_specs=(pl.BlockSpec((_RB, HC), lambda i: (i, 0)),
                   pl.BlockSpec((_RB, HC), lambda i: (i, 0))),
        out_shape=(jax.ShapeDtypeStruct((n, HC), jnp.float32),
                   jax.ShapeDtypeStruct((n, HC), jnp.float32)),
    )(h, p["Wl"], p["bl"].reshape(1, HC), p["Wr"], p["br"].reshape(1, HC))


def _self_score_body(xl_ref, xr_ref, mea_ref, we_ref, attf_ref, sel_ref,
                     a_ref):
    ee = lax.dot_general(mea_ref[...], we_ref[...], (((1,), (1,)), ((), ())),
                         preferred_element_type=jnp.float32)
    z = xl_ref[...] + xr_ref[...] + ee
    z = jnp.where(z >= 0.0, z, 0.2 * z)
    za = z * attf_ref[...]
    a_ref[...] = lax.dot_general(za, sel_ref[...], (((1,), (0,)), ((), ())),
                                 preferred_element_type=jnp.float32)


def _self_scores(xl, xr, mean_ea, We, att):
    attf = att.reshape(1, HC)
    sel = jnp.zeros((HC, H), jnp.float32)
    sel = sel.at[jnp.arange(HC), jnp.arange(HC) // C].set(1.0)
    nb = N // _RB
    return pl.pallas_call(
        _self_score_body,
        grid=(nb,),
        in_specs=[
            pl.BlockSpec((_RB, HC), lambda i: (i, 0)),
            pl.BlockSpec((_RB, HC), lambda i: (i, 0)),
            pl.BlockSpec((_RB, 2), lambda i: (i, 0)),
            pl.BlockSpec((HC, 2), lambda i: (0, 0)),
            pl.BlockSpec((1, HC), lambda i: (0, 0)),
            pl.BlockSpec((HC, H), lambda i: (0, 0)),
        ],
        out_specs=pl.BlockSpec((_RB, H), lambda i: (i, 0)),
        out_shape=jax.ShapeDtypeStruct((N, H), jnp.float32),
    )(xl, xr, mean_ea, We, attf, sel)


def _final_body(h_ref, w_ref, b_ref, g_ref, be_ref, wf_ref, bf_ref, o_ref):
    hh = lax.dot_general(h_ref[...], w_ref[...], (((1,), (1,)), ((), ())),
                         preferred_element_type=jnp.float32) + b_ref[...]
    m = jnp.mean(hh, axis=0, keepdims=True)
    v = jnp.mean((hh - m) ** 2, axis=0, keepdims=True)
    hh = jnp.maximum((hh - m) / jnp.sqrt(v + 1e-5) * g_ref[...] + be_ref[...],
                     0.0)
    lg = lax.dot_general(hh, wf_ref[...], (((1,), (1,)), ((), ())),
                         preferred_element_type=jnp.float32) + bf_ref[...]
    mx = jnp.max(lg, axis=1, keepdims=True)
    el = jnp.exp(lg - mx)
    sm = el / jnp.sum(el, axis=1, keepdims=True)
    o_ref[...] = sm[:, 1:]


def _final_stage(h, W, b, g, be, WF, bF):
    return pl.pallas_call(
        _final_body,
        out_shape=jax.ShapeDtypeStruct((N, 1), jnp.float32),
    )(h, W, b.reshape(1, C), g.reshape(1, C), be.reshape(1, C),
      WF, bF.reshape(1, 2))


_NW = 32
_EPW = E // _NW
_CH = 40
_NCHUNK = _EPW // _CH


_CHG = 24
_NPAIR = _EPW // (2 * _CHG)
_TAIL = _EPW - _NPAIR * 2 * _CHG


def _zsum_body(xl_hbm, xr_hbm, src_hbm, dst_hbm, z_hbm,
               sidx, didx, xlb0, xrb0, xlb1, xrb1, sema, semb):
    w = lax.axis_index("s") * 2 + lax.axis_index("c")
    base = w * _EPW
    pltpu.sync_copy(src_hbm.at[pl.ds(base, _EPW)], sidx)
    pltpu.sync_copy(dst_hbm.at[pl.ds(base, _EPW)], didx)

    def fire(o, xlb, xrb, sem):
        pltpu.async_copy(xl_hbm.at[sidx.at[pl.ds(o, _CHG)]], xlb, sem)
        pltpu.async_copy(xr_hbm.at[didx.at[pl.ds(o, _CHG)]], xrb, sem)

    def drain(xlb, sem):
        pltpu.make_async_copy(xl_hbm.at[pl.ds(0, _CHG)], xlb, sem).wait()
        pltpu.make_async_copy(xl_hbm.at[pl.ds(0, _CHG)], xlb, sem).wait()

    def compute(o, xlb, xrb):
        def row(rr, c2):
            for j in range(HC // 16):
                xlb[rr, pl.ds(j * 16, 16)] = (xlb[rr, pl.ds(j * 16, 16)]
                                              + xrb[rr, pl.ds(j * 16, 16)])
            return c2
        lax.fori_loop(0, _CHG, row, 0, unroll=False)
        pltpu.sync_copy(xlb, z_hbm.at[pl.ds(base + o, _CHG)])

    fire(0, xlb0, xrb0, sema)
    fire(_CHG, xlb1, xrb1, semb)
    last_safe = _EPW - _CHG

    def pair(ci2, c):
        o0 = ci2 * 2 * _CHG
        o1 = o0 + _CHG
        drain(xlb0, sema)
        compute(o0, xlb0, xrb0)
        fire(jnp.minimum(o0 + 2 * _CHG, last_safe), xlb0, xrb0, sema)
        drain(xlb1, semb)
        compute(o1, xlb1, xrb1)
        fire(jnp.minimum(o1 + 2 * _CHG, last_safe), xlb1, xrb1, semb)
        return c

    lax.fori_loop(0, _NPAIR, pair, 0, unroll=False)
    drain(xlb0, sema)
    drain(xlb1, semb)

    to = _NPAIR * 2 * _CHG
    pltpu.async_copy(xl_hbm.at[sidx.at[pl.ds(to, _TAIL)]],
                     xlb0.at[pl.ds(0, _TAIL)], sema).wait()
    pltpu.async_copy(xr_hbm.at[didx.at[pl.ds(to, _TAIL)]],
                     xrb0.at[pl.ds(0, _TAIL)], semb).wait()

    def trow(rr, c2):
        for j in range(HC // 16):
            xlb0[rr, pl.ds(j * 16, 16)] = (xlb0[rr, pl.ds(j * 16, 16)]
                                           + xrb0[rr, pl.ds(j * 16, 16)])
        return c2
    lax.fori_loop(0, _TAIL, trow, 0, unroll=False)
    pltpu.sync_copy(xlb0.at[pl.ds(0, _TAIL)], z_hbm.at[pl.ds(base + to, _TAIL)])


_EPAD = E + 128


def _zsum_sc(xl, xr, ssrc, sdst, offs):
    del offs
    mesh = plsc.VectorSubcoreMesh(core_axis_name="c", subcore_axis_name="s")
    k = functools.partial(
        pl.kernel,
        mesh=mesh,
        compiler_params=pltpu.CompilerParams(needs_layout_passes=False),
        out_type=jax.ShapeDtypeStruct((_EPAD, HC), jnp.float32),
        scratch_types=[
            pltpu.VMEM((_EPW,), jnp.int32),
            pltpu.VMEM((_EPW,), jnp.int32),
            pltpu.VMEM((_CHG, HC), jnp.float32),
            pltpu.VMEM((_CHG, HC), jnp.float32),
            pltpu.VMEM((_CHG, HC), jnp.float32),
            pltpu.VMEM((_CHG, HC), jnp.float32),
            pltpu.SemaphoreType.DMA,
            pltpu.SemaphoreType.DMA,
        ],
    )(_zsum_body)
    return k(xl, xr, ssrc, sdst)


_NR = 160
_RN = 64
_RPW = _NR // _NW
_CHS = 24
_CHA = 96
_NPAD = _NR * _RN
_I16 = lambda: lax.iota(jnp.int32, 16)


def _bcast16(v):
    return jnp.full((16,), v, jnp.int32)


def _scalar(ref, i):
    v = plsc.load_gather(ref, [_bcast16(i)])
    return lax.reduce_max(v, axes=(0,))


def _range_bounds(offsbuf, r):
    lo = _scalar(offsbuf, r)
    hi = _scalar(offsbuf, r + 1)
    return lo, hi


def _mea_body(mepk_hbm, offs_hbm, macc_hbm, offsbuf, mbuf, macc, sem):
    w = lax.axis_index("s") * 2 + lax.axis_index("c")
    pltpu.sync_copy(offs_hbm, offsbuf)
    m3 = _I16() < 3
    zeros = jnp.zeros((16,), jnp.float32)
    for half in range(_RPW):
        r = _RPW * w + half
        nodebase = r * _RN
        lo, hi = _range_bounds(offsbuf, r)

        def zrow(n, c):
            plsc.store_scatter(macc, [_bcast16(n), _I16()], zeros)
            return c
        lax.fori_loop(0, _RN, zrow, 0, unroll=False)

        lo8 = (lo // 8) * 8
        nch = (hi - lo8 + _CHS - 1) // _CHS

        def chunk(ci, c):
            base = lo8 + ci * _CHS
            pltpu.sync_copy(mepk_hbm.at[pl.ds(base, _CHS)], mbuf)
            start = jnp.maximum(lo - base, 0)
            cnt = jnp.minimum(hi - base, _CHS)

            def edge(e, c2):
                av = plsc.load_gather(mbuf, [_bcast16(e), _I16()])
                db = plsc.bitcast(
                    plsc.load_gather(mbuf, [_bcast16(e), _bcast16(8)]),
                    jnp.int32)
                row = db - nodebase
                cur = plsc.load_gather(macc, [row, _I16()])
                plsc.store_scatter(macc, [row, _I16()], cur + av, mask=m3)
                return c2
            lax.fori_loop(start, cnt, edge, 0, unroll=False)
            return c
        lax.fori_loop(0, nch, chunk, 0, unroll=False)
        pltpu.sync_copy(macc, macc_hbm.at[pl.ds(nodebase, _RN)])


def _mea_sc(mepk, offs):
    mesh = plsc.VectorSubcoreMesh(core_axis_name="c", subcore_axis_name="s")
    k = functools.partial(
        pl.kernel,
        mesh=mesh,
        out_type=jax.ShapeDtypeStruct((_NPAD, 16), jnp.float32),
        compiler_params=pltpu.CompilerParams(needs_layout_passes=False),
        scratch_types=[
            pltpu.VMEM((168,), jnp.int32),
            pltpu.VMEM((_CHS, 16), jnp.float32),
            pltpu.VMEM((_RN, 16), jnp.float32),
            pltpu.SemaphoreType.DMA,
        ],
    )(_mea_body)
    return k(mepk, offs)


def _sg_body(aepk_hbm, z_hbm, aself_hbm, offs_hbm,
             agg_hbm, aso_hbm, slo_hbm,
             offsbuf, abuf, zbuf, zbuf2, ab3a, ab3b, maxacc, sumacc,
             aselfbuf, asbuf, slbuf, acc, wbuf, sem, sem3a, sem3b):
    w = lax.axis_index("s") * 2 + lax.axis_index("c")
    pltpu.sync_copy(offs_hbm, offsbuf)
    m8 = _I16() < 8
    zeros = jnp.zeros((16,), jnp.float32)

    for half in range(_RPW):
        r = _RPW * w + half
        nodebase = r * _RN
        lo, hi = _range_bounds(offsbuf, r)
        lo8 = (lo // 8) * 8
        nch = (hi - lo8 + _CHS - 1) // _CHS
        ncha = (hi - lo8 + _CHA - 1) // _CHA

        pltpu.sync_copy(aself_hbm.at[pl.ds(nodebase, _RN)], maxacc)
        pltpu.sync_copy(aself_hbm.at[pl.ds(nodebase, _RN)], aselfbuf)

        def zrow(n, c):
            plsc.store_scatter(sumacc, [_bcast16(n), _I16()], zeros)
            for j in range(HC // 16):
                plsc.store_scatter(acc, [_bcast16(n), _I16() + j * 16], zeros)
            return c
        lax.fori_loop(0, _RN, zrow, 0, unroll=False)

        def chunk1(ci, c):
            base = lo8 + ci * _CHA
            pltpu.sync_copy(aepk_hbm.at[pl.ds(base, _CHA)], abuf)
            start = jnp.maximum(lo - base, 0)
            cnt = jnp.minimum(hi - base, _CHA)

            def edge(e, c2):
                av = plsc.load_gather(abuf, [_bcast16(e), _I16()])
                db = plsc.bitcast(
                    plsc.load_gather(abuf, [_bcast16(e), _bcast16(8)]),
                    jnp.int32)
                row = db - nodebase
                cur = plsc.load_gather(maxacc, [row, _I16()])
                plsc.store_scatter(maxacc, [row, _I16()],
                                   jnp.maximum(cur, av), mask=m8)
                return c2
            lax.fori_loop(start, cnt, edge, 0, unroll=False)
            return c
        lax.fori_loop(0, ncha, chunk1, 0, unroll=False)

        def chunk2(ci, c):
            base = lo8 + ci * _CHA
            pltpu.sync_copy(aepk_hbm.at[pl.ds(base, _CHA)], abuf)
            start = jnp.maximum(lo - base, 0)
            cnt = jnp.minimum(hi - base, _CHA)

            def edge(e, c2):
                av = plsc.load_gather(abuf, [_bcast16(e), _I16()])
                db = plsc.bitcast(
                    plsc.load_gather(abuf, [_bcast16(e), _bcast16(8)]),
                    jnp.int32)
                row = db - nodebase
                mx = plsc.load_gather(maxacc, [row, _I16()])
                pv = jnp.exp(av - mx)
                cur = plsc.load_gather(sumacc, [row, _I16()])
                plsc.store_scatter(sumacc, [row, _I16()], cur + pv, mask=m8)
                return c2
            lax.fori_loop(start, cnt, edge, 0, unroll=False)
            return c
        lax.fori_loop(0, ncha, chunk2, 0, unroll=False)

        def node(n, c):
            nb = _bcast16(n)
            aself = plsc.load_gather(aselfbuf, [nb, _I16()])
            mx = plsc.load_gather(maxacc, [nb, _I16()])
            sm = plsc.load_gather(sumacc, [nb, _I16()])
            pself = jnp.exp(aself - mx)
            inv = 1.0 / (sm + pself + 1e-16)
            plsc.store_scatter(sumacc, [nb, _I16()], inv, mask=m8)
            plsc.store_scatter(asbuf, [nb, _I16()], pself * inv)
            plsc.store_scatter(slbuf, [nb, _I16()], sm * inv)
            return c
        lax.fori_loop(0, _RN, node, 0, unroll=False)

        maxoff = lo8 + jnp.maximum(nch - 1, 0) * _CHS

        def fire3(b, zb, ab, sem):
            pltpu.async_copy(z_hbm.at[pl.ds(b, _CHS)], zb, sem)
            pltpu.async_copy(aepk_hbm.at[pl.ds(b, _CHS)], ab, sem)

        def drain3(zb, ab, sem):
            pltpu.make_async_copy(z_hbm.at[pl.ds(0, _CHS)], zb, sem).wait()
            pltpu.make_async_copy(aepk_hbm.at[pl.ds(0, _CHS)], ab, sem).wait()

        def proc3(b, ab, zb):
            start = jnp.maximum(lo - b, 0)
            cnt = jnp.minimum(hi - b, _CHS)

            def edge(e, c2):
                eb = _bcast16(e)
                db = plsc.bitcast(
                    plsc.load_gather(ab, [eb, _bcast16(8)]), jnp.int32)
                row = db - nodebase
                ws = []
                for h in range(H):
                    hv = _bcast16(h)
                    avh = plsc.load_gather(ab, [eb, hv])
                    mh = plsc.load_gather(maxacc, [row, hv])
                    ivh = plsc.load_gather(sumacc, [row, hv])
                    ws.append(jnp.exp(avh - mh) * ivh)
                for j in range(HC // 16):
                    col = _I16() + j * 16
                    zv = plsc.load_gather(zb, [eb, col])
                    plsc.addupdate_scatter(acc, [row, col], zv * ws[j // 4])
                return c2
            lax.fori_loop(start, cnt, edge, 0, unroll=False)

        fire3(lo8, zbuf, ab3a, sem3a)
        fire3(jnp.minimum(lo8 + _CHS, maxoff), zbuf2, ab3b, sem3b)
        npair3 = (nch + 1) // 2

        def pair3(ci2, c):
            b0 = lo8 + (2 * ci2) * _CHS
            b1 = b0 + _CHS
            drain3(zbuf, ab3a, sem3a)
            proc3(b0, ab3a, zbuf)
            fire3(jnp.minimum(b0 + 2 * _CHS, maxoff), zbuf, ab3a, sem3a)
            drain3(zbuf2, ab3b, sem3b)
            proc3(b1, ab3b, zbuf2)
            fire3(jnp.minimum(b1 + 2 * _CHS, maxoff), zbuf2, ab3b, sem3b)
            return c

        lax.fori_loop(0, npair3, pair3, 0, unroll=False)
        drain3(zbuf, ab3a, sem3a)
        drain3(zbuf2, ab3b, sem3b)

        pltpu.sync_copy(acc, agg_hbm.at[pl.ds(nodebase, _RN)])
        pltpu.sync_copy(asbuf, aso_hbm.at[pl.ds(nodebase, _RN)])
        pltpu.sync_copy(slbuf, slo_hbm.at[pl.ds(nodebase, _RN)])


def _sg_sc(aepk, zsum, aselfpk, offs):
    mesh = plsc.VectorSubcoreMesh(core_axis_name="c", subcore_axis_name="s")
    k = functools.partial(
        pl.kernel,
        mesh=mesh,
        compiler_params=pltpu.CompilerParams(needs_layout_passes=False),
        out_type=(jax.ShapeDtypeStruct((_NPAD, HC), jnp.float32),
                  jax.ShapeDtypeStruct((_NPAD, 16), jnp.float32),
                  jax.ShapeDtypeStruct((_NPAD, 16), jnp.float32)),
        scratch_types=[
            pltpu.VMEM((168,), jnp.int32),
            pltpu.VMEM((_CHA, 16), jnp.float32),
            pltpu.VMEM((_CHS, HC), jnp.float32),
            pltpu.VMEM((_CHS, HC), jnp.float32),
            pltpu.VMEM((_CHS, 16), jnp.float32),
            pltpu.VMEM((_CHS, 16), jnp.float32),
            pltpu.VMEM((_RN, 16), jnp.float32),
            pltpu.VMEM((_RN, 16), jnp.float32),
            pltpu.VMEM((_RN, 16), jnp.float32),
            pltpu.VMEM((_RN, 16), jnp.float32),
            pltpu.VMEM((_RN, 16), jnp.float32),
            pltpu.VMEM((_RN, HC), jnp.float32),
            pltpu.VMEM((16,), jnp.float32),
            pltpu.SemaphoreType.DMA,
            pltpu.SemaphoreType.DMA,
            pltpu.SemaphoreType.DMA,
        ],
    )(_sg_body)
    return k(aepk, zsum, aselfpk, offs)


_EB = 2000


def _escore_body(z_ref, ea_ref, we_ref, attf_ref, sel_ref, m_ref, d_ref,
                 o_ref):
    ee = lax.dot_general(ea_ref[...], we_ref[...], (((1,), (1,)), ((), ())),
                         preferred_element_type=jnp.float32)
    z = z_ref[...] + ee
    z = jnp.where(z >= 0.0, z, 0.2 * z)
    a = lax.dot_general(z * attf_ref[...], sel_ref[...],
                        (((1,), (0,)), ((), ())),
                        preferred_element_type=jnp.float32)
    a = jnp.where(m_ref[...] > 0.0, a, -jnp.inf)
    zpad = jnp.zeros((a.shape[0], 7), jnp.float32)
    o_ref[...] = jnp.concatenate([a, d_ref[...], zpad], axis=1)


def _edge_scores(zsum, sea, We, att, maskf, sdstf):
    attf = att.reshape(1, HC)
    sel = jnp.zeros((HC, H), jnp.float32)
    sel = sel.at[jnp.arange(HC), jnp.arange(HC) // C].set(1.0)
    nb = E // _EB
    return pl.pallas_call(
        _escore_body,
        grid=(nb,),
        in_specs=[
            pl.BlockSpec((_EB, HC), lambda i: (i, 0)),
            pl.BlockSpec((_EB, 2), lambda i: (i, 0)),
            pl.BlockSpec((HC, 2), lambda i: (0, 0)),
            pl.BlockSpec((1, HC), lambda i: (0, 0)),
            pl.BlockSpec((HC, H), lambda i: (0, 0)),
            pl.BlockSpec((_EB, 1), lambda i: (i, 0)),
            pl.BlockSpec((_EB, 1), lambda i: (i, 0)),
        ],
        out_specs=pl.BlockSpec((_EB, 16), lambda i: (i, 0)),
        out_shape=jax.ShapeDtypeStruct((E, 16), jnp.float32),
    )(zsum, sea, We, attf, sel, maskf.reshape(E, 1), sdstf.reshape(E, 1))


def _combine_body(agg_ref, xl_ref, xr_ref, as_ref, sl_ref, b_ref, exp_ref,
                  o_ref):
    aexp = lax.dot_general(as_ref[...], exp_ref[...], (((1,), (0,)), ((), ())),
                           preferred_element_type=jnp.float32)
    sexp = lax.dot_general(sl_ref[...], exp_ref[...], (((1,), (0,)), ((), ())),
                           preferred_element_type=jnp.float32)
    out = (agg_ref[...] - sexp * xr_ref[...] + aexp * xl_ref[...]
           + b_ref[...])
    o_ref[...] = jnp.maximum(out, 0.0)


def _combine(agg, xl, xr, aself_o, salpha_o, bias):
    exp16 = jnp.zeros((16, HC), jnp.float32)
    exp16 = exp16.at[jnp.arange(HC) // C, jnp.arange(HC)].set(1.0)
    nb = N // _RB
    return pl.pallas_call(
        _combine_body,
        grid=(nb,),
        in_specs=[
            pl.BlockSpec((_RB, HC), lambda i: (i, 0)),
            pl.BlockSpec((_RB, HC), lambda i: (i, 0)),
            pl.BlockSpec((_RB, HC), lambda i: (i, 0)),
            pl.BlockSpec((_RB, 16), lambda i: (i, 0)),
            pl.BlockSpec((_RB, 16), lambda i: (i, 0)),
            pl.BlockSpec((1, HC), lambda i: (0, 0)),
            pl.BlockSpec((16, HC), lambda i: (0, 0)),
        ],
        out_specs=pl.BlockSpec((_RB, HC), lambda i: (i, 0)),
        out_shape=jax.ShapeDtypeStruct((N, HC), jnp.float32),
    )(agg, xl, xr, aself_o, salpha_o, bias.reshape(1, HC), exp16)


def _gat_layer(h, ssrc, sdst, sdstf, sea, offs, p, smaskf):
    xl, xr = _gat_proj(h, p)

    mea = sea * smaskf[:, None]
    mepk = jnp.concatenate(
        [mea, smaskf[:, None], jnp.zeros((E, 5), jnp.float32),
         sdstf[:, None], jnp.zeros((E, 7), jnp.float32)], axis=1)
    mepk = jnp.concatenate([mepk, jnp.zeros((128, 16), jnp.float32)], axis=0)
    macc = _mea_sc(mepk, offs)
    mean_ea = macc[:N, 0:2] / jnp.maximum(macc[:N, 2:3], 1.0)

    a_self = _self_scores(xl, xr, mean_ea, p["We"], p["att"])
    aselfpk = jnp.pad(a_self, ((0, _NPAD - N), (0, 8)))

    zsum = _zsum_sc(xl, xr, ssrc, sdst, offs)
    a16 = _edge_scores(zsum, sea, p["We"], p["att"], smaskf, sdstf)
    aepk = jnp.concatenate([a16, jnp.zeros((128, 16), jnp.float32)], axis=0)

    agg, aself_o, salpha_o = _sg_sc(aepk, zsum, aselfpk, offs)

    return _combine(agg, xl, xr, aself_o, salpha_o, p["bias"])


def kernel(x, edge_index, edge_attr, params):
    src = edge_index[0]
    dst = edge_index[1]
    a0 = edge_attr[:, 0]
    fea = edge_attr[:, 1:]
    p = params

    perm = jnp.argsort(dst)
    ssrc = src[perm]
    sdst = dst[perm]
    sea = fea[perm]
    sa0 = a0[perm]
    sdstf = lax.bitcast_convert_type(sdst, jnp.float32)
    bounds = jnp.minimum(jnp.arange(_NR + 1, dtype=jnp.int32) * _RN, N)
    offs = jnp.searchsorted(sdst, bounds, side="left").astype(jnp.int32)
    offs = jnp.concatenate([offs, jnp.zeros((168 - _NR - 1,), jnp.int32)])
    m1 = (sa0 >= 0).astype(jnp.float32)
    m2 = (sa0 <= 0).astype(jnp.float32)
    mall = jnp.ones((E,), jnp.float32)

    h = _lin_bn_relu(x, p["W0"], p["b0"], p["g0"], p["be0"])
    h = _gat_layer(h, ssrc, sdst, sdstf, sea, offs, p["gat1"], m1)
    h = _lin_bn_relu(h, p["W1"], p["b1"], p["g1"], p["be1"])
    h = _gat_layer(h, ssrc, sdst, sdstf, sea, offs, p["gat2"], m2)
    h = _lin_bn_relu(h, p["W2"], p["b2"], p["g2"], p["be2"])
    h = _gat_layer(h, ssrc, sdst, sdstf, sea, offs, p["gat3"], mall)
    return _final_stage(h, p["W3"], p["b3"], p["g3"], p["be3"],
                        p["WF"], p["bF"])
